# Initial kernel scaffold; baseline (speedup 1.0000x reference)
import jax
import jax.numpy as jnp
from jax.experimental import pallas as pl


def kernel(x, edge_index, batch, emb_table, W1, b1, W2, b2, Wlin, blin):
    raise NotImplementedError("write your pallas kernel here")


# speedup vs baseline: 7.8766x; 7.8766x over previous
import functools

import jax
import jax.numpy as jnp
from jax import lax
from jax.experimental import pallas as pl
from jax.experimental.pallas import tpu as pltpu
from jax.experimental.pallas import tpu_sc as plsc

N_NODES = 10000
N_EDGES = 320000
VOCAB = 1000
DIM = 128
NUM_CLASSES = 10
NUM_GRAPHS = 64

NC, NS = 2, 16
NW = NC * NS
CH = 128
CPW = 80
E_PAD = NW * CPW * CH
NP = 10240
N_NCH = NP // CH
RPS = NP // NS

_mesh = plsc.VectorSubcoreMesh(
    core_axis_name="c", subcore_axis_name="s", num_cores=NC, num_subcores=NS)


def _wid():
  return lax.axis_index("s") * NC + lax.axis_index("c")


@functools.partial(
    pl.kernel,
    out_type=(
        jax.ShapeDtypeStruct((NP,), jnp.float32),
        jax.ShapeDtypeStruct((NP,), jnp.float32),
        jax.ShapeDtypeStruct((NP, DIM), jnp.float32),
    ),
    mesh=_mesh,
    scratch_types=(
        pltpu.VMEM((CPW, CH), jnp.int32),
        pltpu.VMEM((CH,), jnp.float32),
        pltpu.VMEM((CH,), jnp.int32),
        pltpu.VMEM((CH, DIM), jnp.float32),
        pltpu.VMEM_SHARED((NP,), jnp.float32),
    ),
)
def _sc_deg_embed(dst_hbm, ones_hbm, table_hbm, x_hbm, deg0_out, deg1_out,
                  h0_out, dst_v, ones_v, x_v, rows_v, accd):
  c = lax.axis_index("c")
  s = lax.axis_index("s")
  w = _wid()

  pltpu.sync_copy(ones_hbm.at[pl.ds(0, RPS)], accd.at[pl.ds(s * RPS, RPS)])
  pltpu.sync_copy(ones_hbm.at[pl.ds(0, CH)], ones_v)
  pltpu.sync_copy(dst_hbm.at[w], dst_v)
  plsc.subcore_barrier()

  @pl.loop(0, CPW)
  def _count(j):
    pltpu.sync_copy(ones_v, accd.at[dst_v.at[j]], add=True)

  @pl.loop(w, N_NCH, step=NW)
  def _embed(t):
    pltpu.sync_copy(x_hbm.at[pl.ds(t * CH, CH)], x_v)
    pltpu.sync_copy(table_hbm.at[x_v], rows_v)
    pltpu.sync_copy(rows_v, h0_out.at[pl.ds(t * CH, CH)])

  plsc.subcore_barrier()

  @pl.when(c == 0)
  def _():
    pltpu.sync_copy(accd.at[pl.ds(s * RPS, RPS)],
                    deg0_out.at[pl.ds(s * RPS, RPS)])

  @pl.when(c == 1)
  def _():
    pltpu.sync_copy(accd.at[pl.ds(s * RPS, RPS)],
                    deg1_out.at[pl.ds(s * RPS, RPS)])


@functools.partial(
    pl.kernel,
    out_type=jax.ShapeDtypeStruct((NC, NP, DIM), jnp.float32),
    mesh=_mesh,
    scratch_types=(
        pltpu.VMEM((CPW * CH,), jnp.int32),
        pltpu.VMEM((CPW, CH), jnp.int32),
        pltpu.VMEM((CH, DIM), jnp.float32),
        pltpu.VMEM_SHARED((NP, DIM), jnp.float32),
    ),
)
def _sc_edge_agg(src_hbm, dst_hbm, g_hbm, acc_out, src_v, dst_v, rows_v, acc):
  c = lax.axis_index("c")
  s = lax.axis_index("s")
  w = _wid()

  pltpu.sync_copy(g_hbm.at[pl.ds(s * RPS, RPS)], acc.at[pl.ds(s * RPS, RPS)])
  pltpu.sync_copy(src_hbm.at[pl.ds(w * CPW * CH, CPW * CH)], src_v)
  pltpu.sync_copy(dst_hbm.at[w], dst_v)
  plsc.subcore_barrier()

  @pl.loop(0, CPW)
  def _agg(j):
    pltpu.sync_copy(g_hbm.at[src_v.at[pl.ds(j * CH, CH)]], rows_v)
    pltpu.sync_copy(rows_v, acc.at[dst_v.at[j]], add=True)

  plsc.subcore_barrier()
  pltpu.sync_copy(acc.at[pl.ds(s * RPS, RPS)],
                  acc_out.at[c].at[pl.ds(s * RPS, RPS)])


def _tc_table_body(emb_ref, out_ref):
  rows = lax.broadcasted_iota(jnp.int32, (VOCAB, DIM), 0)
  out_ref[...] = jnp.where(rows == 0, 0.0, emb_ref[...])


def _dinv(deg0_ref, deg1_ref):
  deg = deg0_ref[...] + deg1_ref[...] - 1.0
  return lax.rsqrt(deg)


def _tc_g1_body(h0_ref, deg0_ref, deg1_ref, w1_ref, g1_ref):
  d = _dinv(deg0_ref, deg1_ref)
  g1_ref[...] = d * jnp.dot(h0_ref[...], w1_ref[...],
                            preferred_element_type=jnp.float32)


def _tc_g2_body(p_ref, g1_ref, deg0_ref, deg1_ref, b1_ref, w2_ref, g2_ref):
  d = _dinv(deg0_ref, deg1_ref)
  p = p_ref[...]
  p_raw = p[0] + p[1] - g1_ref[...]
  h1 = jnp.maximum(d * p_raw + b1_ref[...], 0.0)
  g2_ref[...] = d * jnp.dot(h1, w2_ref[...],
                            preferred_element_type=jnp.float32)


def _tc_final_body(q_ref, g2_ref, deg0_ref, deg1_ref, b2_ref, batch_ref,
                   wlin_ref, blin_ref, out_ref):
  d = _dinv(deg0_ref, deg1_ref)
  q = q_ref[...]
  p_raw = q[0] + q[1] - g2_ref[...]
  h2 = jnp.maximum(d * p_raw + b2_ref[...], 0.0)
  gid = lax.broadcasted_iota(jnp.int32, (1, NUM_GRAPHS), 1)
  onehot = (batch_ref[...] == gid).astype(jnp.float32)
  cnt = jnp.sum(onehot, axis=0, keepdims=True)
  pooled = lax.dot_general(onehot, h2, (((0,), (0,)), ((), ())),
                           preferred_element_type=jnp.float32)
  pooled = pooled / jnp.maximum(cnt, 1.0).T
  out_ref[...] = jnp.dot(pooled, wlin_ref[...],
                         preferred_element_type=jnp.float32) + blin_ref[...]


_tc_table = pl.pallas_call(
    _tc_table_body,
    out_shape=jax.ShapeDtypeStruct((VOCAB, DIM), jnp.float32))

_tc_g1 = pl.pallas_call(
    _tc_g1_body,
    out_shape=jax.ShapeDtypeStruct((NP, DIM), jnp.float32))

_tc_g2 = pl.pallas_call(
    _tc_g2_body,
    out_shape=jax.ShapeDtypeStruct((NP, DIM), jnp.float32))

_tc_final = pl.pallas_call(
    _tc_final_body,
    out_shape=jax.ShapeDtypeStruct((NUM_GRAPHS, NUM_CLASSES), jnp.float32))


@jax.jit
def kernel(x, edge_index, batch, emb_table, W1, b1, W2, b2, Wlin, blin):
  x = x.astype(jnp.int32)
  pad_e = E_PAD - N_EDGES
  pad_n = NP - N_NODES
  src = jnp.concatenate([edge_index[0], jnp.zeros((pad_e,), jnp.int32)])
  dst = jnp.concatenate(
      [edge_index[1],
       N_NODES + (jnp.arange(pad_e, dtype=jnp.int32) % pad_n)]
  ).reshape(NW, CPW, CH)
  xp = jnp.concatenate([x, jnp.zeros((pad_n,), jnp.int32)])
  batchp = jnp.concatenate(
      [batch, jnp.full((pad_n,), NUM_GRAPHS, jnp.int32)]).reshape(NP, 1)
  ones_c = jnp.ones((RPS,), jnp.float32)

  table_z = _tc_table(emb_table)
  deg0, deg1, h0 = _sc_deg_embed(dst, ones_c, table_z, xp)
  deg0 = deg0.reshape(NP, 1)
  deg1 = deg1.reshape(NP, 1)
  g1 = _tc_g1(h0, deg0, deg1, W1)
  p1 = _sc_edge_agg(src, dst, g1)
  g2 = _tc_g2(p1, g1, deg0, deg1, b1.reshape(1, DIM), W2)
  p2 = _sc_edge_agg(src, dst, g2)
  return _tc_final(p2, g2, deg0, deg1, b2.reshape(1, DIM), batchp, Wlin,
                   blin.reshape(1, NUM_CLASSES))


# speedup vs baseline: 8.4834x; 1.0770x over previous
import functools

import jax
import jax.numpy as jnp
from jax import lax
from jax.experimental import pallas as pl
from jax.experimental.pallas import tpu as pltpu
from jax.experimental.pallas import tpu_sc as plsc

N_NODES = 10000
N_EDGES = 320000
VOCAB = 1000
DIM = 128
NUM_CLASSES = 10
NUM_GRAPHS = 64

NC, NS = 2, 16
NW = NC * NS
CH = 128
CPW = 80
E_PAD = NW * CPW * CH
NP = 10240
N_NCH = NP // CH
RPS = NP // NS

_mesh = plsc.VectorSubcoreMesh(
    core_axis_name="c", subcore_axis_name="s", num_cores=NC, num_subcores=NS)


def _wid():
  return lax.axis_index("s") * NC + lax.axis_index("c")


@functools.partial(
    pl.kernel,
    out_type=(
        jax.ShapeDtypeStruct((NP,), jnp.float32),
        jax.ShapeDtypeStruct((NP,), jnp.float32),
        jax.ShapeDtypeStruct((NP, DIM), jnp.float32),
    ),
    mesh=_mesh,
    scratch_types=(
        pltpu.VMEM((CPW, CH), jnp.int32),
        pltpu.VMEM((CH,), jnp.float32),
        pltpu.VMEM((CH,), jnp.int32),
        pltpu.VMEM((CH, DIM), jnp.float32),
        pltpu.VMEM_SHARED((NP,), jnp.float32),
    ),
)
def _sc_deg_embed(dst_hbm, ones_hbm, table_hbm, x_hbm, deg0_out, deg1_out,
                  h0_out, dst_v, ones_v, x_v, rows_v, accd):
  c = lax.axis_index("c")
  s = lax.axis_index("s")
  w = _wid()

  pltpu.sync_copy(ones_hbm.at[pl.ds(0, RPS)], accd.at[pl.ds(s * RPS, RPS)])
  pltpu.sync_copy(ones_hbm.at[pl.ds(0, CH)], ones_v)
  pltpu.sync_copy(dst_hbm.at[w], dst_v)
  plsc.subcore_barrier()

  @pl.loop(0, CPW)
  def _count(j):
    pltpu.sync_copy(ones_v, accd.at[dst_v.at[j]], add=True)

  @pl.loop(w, N_NCH, step=NW)
  def _embed(t):
    pltpu.sync_copy(x_hbm.at[pl.ds(t * CH, CH)], x_v)
    pltpu.sync_copy(table_hbm.at[x_v], rows_v)
    pltpu.sync_copy(rows_v, h0_out.at[pl.ds(t * CH, CH)])

  plsc.subcore_barrier()

  @pl.when(c == 0)
  def _():
    pltpu.sync_copy(accd.at[pl.ds(s * RPS, RPS)],
                    deg0_out.at[pl.ds(s * RPS, RPS)])

  @pl.when(c == 1)
  def _():
    pltpu.sync_copy(accd.at[pl.ds(s * RPS, RPS)],
                    deg1_out.at[pl.ds(s * RPS, RPS)])


NBUF = 2
NH = 2
HCPW = CPW // NH
HGRP = HCPW // NBUF


@functools.partial(
    pl.kernel,
    out_type=jax.ShapeDtypeStruct((NC, NP, DIM), jnp.float32),
    mesh=_mesh,
    scratch_types=(
        pltpu.VMEM((HCPW * CH,), jnp.int32),
        pltpu.VMEM((HCPW, CH), jnp.int32),
        tuple(pltpu.VMEM((CH, DIM), jnp.float32) for _ in range(NBUF)),
        tuple(pltpu.SemaphoreType.DMA for _ in range(NBUF)),
        tuple(pltpu.SemaphoreType.DMA for _ in range(NBUF)),
        pltpu.VMEM_SHARED((NP, DIM), jnp.float32),
    ),
)
def _sc_edge_agg(src_hbm, dst_hbm, g_hbm, acc_out, src_v, dst_v, rows,
                 gsem, ssem, acc):
  c = lax.axis_index("c")
  s = lax.axis_index("s")
  w = _wid()

  pltpu.sync_copy(g_hbm.at[pl.ds(s * RPS, RPS)], acc.at[pl.ds(s * RPS, RPS)])
  plsc.subcore_barrier()

  def gather(j, b):
    pltpu.async_copy(g_hbm.at[src_v.at[pl.ds(j * CH, CH)]], rows[b], gsem[b])

  def gather_wait(b):
    pltpu.make_async_copy(g_hbm.at[pl.ds(0, CH)], rows[b], gsem[b]).wait()

  def scatter(j, b):
    pltpu.async_copy(rows[b], acc.at[dst_v.at[j]], ssem[b], add=True)

  def scatter_wait(b):
    pltpu.make_async_copy(rows[b], acc.at[pl.ds(0, CH)], ssem[b]).wait()

  for h in range(NH):
    pltpu.sync_copy(
        src_hbm.at[pl.ds((w * CPW + h * HCPW) * CH, HCPW * CH)], src_v)
    pltpu.sync_copy(dst_hbm.at[w].at[pl.ds(h * HCPW, HCPW)], dst_v)

    for b in range(NBUF):
      gather(b, b)

    @pl.loop(0, HGRP - 1)
    def _agg(i):
      base = i * NBUF
      for b in range(NBUF):
        gather_wait(b)
        scatter(base + b, b)
      for b in range(NBUF):
        scatter_wait(b)
        gather(base + NBUF + b, b)

    for b in range(NBUF):
      gather_wait(b)
      scatter(HCPW - NBUF + b, b)
    for b in range(NBUF):
      scatter_wait(b)

  plsc.subcore_barrier()
  pltpu.sync_copy(acc.at[pl.ds(s * RPS, RPS)],
                  acc_out.at[c].at[pl.ds(s * RPS, RPS)])


def _tc_table_body(emb_ref, out_ref):
  rows = lax.broadcasted_iota(jnp.int32, (VOCAB, DIM), 0)
  out_ref[...] = jnp.where(rows == 0, 0.0, emb_ref[...])


def _dinv(deg0_ref, deg1_ref):
  deg = deg0_ref[...] + deg1_ref[...] - 1.0
  return lax.rsqrt(deg)


def _tc_g1_body(h0_ref, deg0_ref, deg1_ref, w1_ref, g1_ref):
  d = _dinv(deg0_ref, deg1_ref)
  g1_ref[...] = d * jnp.dot(h0_ref[...], w1_ref[...],
                            preferred_element_type=jnp.float32)


def _tc_g2_body(p_ref, g1_ref, deg0_ref, deg1_ref, b1_ref, w2_ref, g2_ref):
  d = _dinv(deg0_ref, deg1_ref)
  p = p_ref[...]
  p_raw = p[0] + p[1] - g1_ref[...]
  h1 = jnp.maximum(d * p_raw + b1_ref[...], 0.0)
  g2_ref[...] = d * jnp.dot(h1, w2_ref[...],
                            preferred_element_type=jnp.float32)


def _tc_final_body(q_ref, g2_ref, deg0_ref, deg1_ref, b2_ref, batch_ref,
                   wlin_ref, blin_ref, out_ref):
  d = _dinv(deg0_ref, deg1_ref)
  q = q_ref[...]
  p_raw = q[0] + q[1] - g2_ref[...]
  h2 = jnp.maximum(d * p_raw + b2_ref[...], 0.0)
  gid = lax.broadcasted_iota(jnp.int32, (1, NUM_GRAPHS), 1)
  onehot = (batch_ref[...] == gid).astype(jnp.float32)
  cnt = jnp.sum(onehot, axis=0, keepdims=True)
  pooled = lax.dot_general(onehot, h2, (((0,), (0,)), ((), ())),
                           preferred_element_type=jnp.float32)
  pooled = pooled / jnp.maximum(cnt, 1.0).T
  out_ref[...] = jnp.dot(pooled, wlin_ref[...],
                         preferred_element_type=jnp.float32) + blin_ref[...]


_tc_table = pl.pallas_call(
    _tc_table_body,
    out_shape=jax.ShapeDtypeStruct((VOCAB, DIM), jnp.float32))

_tc_g1 = pl.pallas_call(
    _tc_g1_body,
    out_shape=jax.ShapeDtypeStruct((NP, DIM), jnp.float32))

_tc_g2 = pl.pallas_call(
    _tc_g2_body,
    out_shape=jax.ShapeDtypeStruct((NP, DIM), jnp.float32))

_tc_final = pl.pallas_call(
    _tc_final_body,
    out_shape=jax.ShapeDtypeStruct((NUM_GRAPHS, NUM_CLASSES), jnp.float32))


@jax.jit
def kernel(x, edge_index, batch, emb_table, W1, b1, W2, b2, Wlin, blin):
  x = x.astype(jnp.int32)
  pad_e = E_PAD - N_EDGES
  pad_n = NP - N_NODES
  src = jnp.concatenate([edge_index[0], jnp.zeros((pad_e,), jnp.int32)])
  dst = jnp.concatenate(
      [edge_index[1],
       N_NODES + (jnp.arange(pad_e, dtype=jnp.int32) % pad_n)]
  ).reshape(NW, CPW, CH)
  xp = jnp.concatenate([x, jnp.zeros((pad_n,), jnp.int32)])
  batchp = jnp.concatenate(
      [batch, jnp.full((pad_n,), NUM_GRAPHS, jnp.int32)]).reshape(NP, 1)
  ones_c = jnp.ones((RPS,), jnp.float32)

  table_z = _tc_table(emb_table)
  deg0, deg1, h0 = _sc_deg_embed(dst, ones_c, table_z, xp)
  deg0 = deg0.reshape(NP, 1)
  deg1 = deg1.reshape(NP, 1)
  g1 = _tc_g1(h0, deg0, deg1, W1)
  p1 = _sc_edge_agg(src, dst, g1)
  g2 = _tc_g2(p1, g1, deg0, deg1, b1.reshape(1, DIM), W2)
  p2 = _sc_edge_agg(src, dst, g2)
  return _tc_final(p2, g2, deg0, deg1, b2.reshape(1, DIM), batchp, Wlin,
                   blin.reshape(1, NUM_CLASSES))


# speedup vs baseline: 22.8180x; 2.6897x over previous
import functools

import jax
import jax.numpy as jnp
from jax import lax
from jax.experimental import pallas as pl
from jax.experimental.pallas import tpu as pltpu
from jax.experimental.pallas import tpu_sc as plsc

N_NODES = 10000
N_EDGES = 320000
VOCAB = 1000
DIM = 128
NUM_CLASSES = 10
NUM_GRAPHS = 64

NC, NS = 2, 16
NW = NC * NS
CH = 128
CPW = 80
E_PAD = NW * CPW * CH
NP = 10240
N_NCH = NP // CH
RPS = NP // NS

_mesh = plsc.VectorSubcoreMesh(
    core_axis_name="c", subcore_axis_name="s", num_cores=NC, num_subcores=NS)


def _wid():
  return lax.axis_index("s") * NC + lax.axis_index("c")


@functools.partial(
    pl.kernel,
    out_type=(
        jax.ShapeDtypeStruct((NP,), jnp.float32),
        jax.ShapeDtypeStruct((NP,), jnp.float32),
        jax.ShapeDtypeStruct((NP, DIM), jnp.float32),
    ),
    mesh=_mesh,
    scratch_types=(
        pltpu.VMEM((CPW, CH), jnp.int32),
        pltpu.VMEM((CH,), jnp.float32),
        pltpu.VMEM((CH,), jnp.int32),
        pltpu.VMEM((CH, DIM), jnp.float32),
        pltpu.VMEM_SHARED((NP,), jnp.float32),
    ),
)
def _sc_deg_embed(dst_hbm, ones_hbm, table_hbm, x_hbm, deg0_out, deg1_out,
                  h0_out, dst_v, ones_v, x_v, rows_v, accd):
  c = lax.axis_index("c")
  s = lax.axis_index("s")
  w = _wid()

  pltpu.sync_copy(ones_hbm.at[pl.ds(0, RPS)], accd.at[pl.ds(s * RPS, RPS)])
  pltpu.sync_copy(ones_hbm.at[pl.ds(0, CH)], ones_v)
  pltpu.sync_copy(dst_hbm.at[w], dst_v)
  plsc.subcore_barrier()

  @pl.loop(0, CPW)
  def _count(j):
    pltpu.sync_copy(ones_v, accd.at[dst_v.at[j]], add=True)

  @pl.loop(w, N_NCH, step=NW)
  def _embed(t):
    pltpu.sync_copy(x_hbm.at[pl.ds(t * CH, CH)], x_v)
    pltpu.sync_copy(table_hbm.at[x_v], rows_v)
    pltpu.sync_copy(rows_v, h0_out.at[pl.ds(t * CH, CH)])

  plsc.subcore_barrier()

  @pl.when(c == 0)
  def _():
    pltpu.sync_copy(accd.at[pl.ds(s * RPS, RPS)],
                    deg0_out.at[pl.ds(s * RPS, RPS)])

  @pl.when(c == 1)
  def _():
    pltpu.sync_copy(accd.at[pl.ds(s * RPS, RPS)],
                    deg1_out.at[pl.ds(s * RPS, RPS)])


NBUF = 2
NH = 2
HCPW = CPW // NH
HGRP = HCPW // NBUF


@functools.partial(
    pl.kernel,
    out_type=jax.ShapeDtypeStruct((NC, NP, DIM), jnp.float32),
    mesh=_mesh,
    scratch_types=(
        pltpu.VMEM((HCPW * CH,), jnp.int32),
        pltpu.VMEM((HCPW, CH), jnp.int32),
        tuple(pltpu.VMEM((CH, DIM), jnp.float32) for _ in range(NBUF)),
        tuple(pltpu.SemaphoreType.DMA for _ in range(NBUF)),
        tuple(pltpu.SemaphoreType.DMA for _ in range(NBUF)),
        pltpu.VMEM_SHARED((NP, DIM), jnp.float32),
    ),
)
def _sc_edge_agg(src_hbm, dst_hbm, g_hbm, acc_out, src_v, dst_v, rows,
                 gsem, ssem, acc):
  c = lax.axis_index("c")
  s = lax.axis_index("s")
  w = _wid()

  pltpu.sync_copy(g_hbm.at[pl.ds(s * RPS, RPS)], acc.at[pl.ds(s * RPS, RPS)])
  plsc.subcore_barrier()

  def gather(j, b):
    pltpu.async_copy(g_hbm.at[src_v.at[pl.ds(j * CH, CH)]], rows[b], gsem[b])

  def gather_wait(b):
    pltpu.make_async_copy(g_hbm.at[pl.ds(0, CH)], rows[b], gsem[b]).wait()

  def scatter(j, b):
    pltpu.async_copy(rows[b], acc.at[dst_v.at[j]], ssem[b], add=True)

  def scatter_wait(b):
    pltpu.make_async_copy(rows[b], acc.at[pl.ds(0, CH)], ssem[b]).wait()

  for h in range(NH):
    pltpu.sync_copy(
        src_hbm.at[pl.ds((w * CPW + h * HCPW) * CH, HCPW * CH)], src_v)
    pltpu.sync_copy(dst_hbm.at[w].at[pl.ds(h * HCPW, HCPW)], dst_v)

    for b in range(NBUF):
      gather(b, b)

    @pl.loop(0, HGRP - 1)
    def _agg(i):
      base = i * NBUF
      for b in range(NBUF):
        gather_wait(b)
        scatter(base + b, b)
      for b in range(NBUF):
        scatter_wait(b)
        gather(base + NBUF + b, b)

    for b in range(NBUF):
      gather_wait(b)
      scatter(HCPW - NBUF + b, b)
    for b in range(NBUF):
      scatter_wait(b)

  plsc.subcore_barrier()
  pltpu.sync_copy(acc.at[pl.ds(s * RPS, RPS)],
                  acc_out.at[c].at[pl.ds(s * RPS, RPS)])


def _tc_table_body(emb_ref, out_ref):
  rows = lax.broadcasted_iota(jnp.int32, (VOCAB, DIM), 0)
  out_ref[...] = jnp.where(rows == 0, 0.0, emb_ref[...])


def _dinv(deg0_ref, deg1_ref):
  deg = deg0_ref[...] + deg1_ref[...] - 1.0
  return lax.rsqrt(deg)


def _rowmask():
  return (lax.broadcasted_iota(jnp.int32, (NP, 1), 0) < N_NODES).astype(
      jnp.float32)


def _tc_g1_body(h0_ref, deg0_ref, deg1_ref, w1_ref, g1_ref):
  d = _dinv(deg0_ref, deg1_ref) * _rowmask()
  g1_ref[...] = d * jnp.dot(h0_ref[...], w1_ref[...],
                            preferred_element_type=jnp.float32)


def _tc_g2_body(p_ref, g1_ref, deg0_ref, deg1_ref, b1_ref, w2_ref, g2_ref):
  d = _dinv(deg0_ref, deg1_ref)
  p = p_ref[...]
  p_raw = p[0] + p[1] - g1_ref[...]
  h1 = jnp.maximum(d * p_raw + b1_ref[...], 0.0)
  g2_ref[...] = (d * _rowmask()) * jnp.dot(h1, w2_ref[...],
                                           preferred_element_type=jnp.float32)


def _tc_final_body(q_ref, g2_ref, deg0_ref, deg1_ref, b2_ref, batch_ref,
                   wlin_ref, blin_ref, out_ref):
  d = _dinv(deg0_ref, deg1_ref)
  q = q_ref[...]
  p_raw = q[0] + q[1] - g2_ref[...]
  h2 = jnp.maximum(d * p_raw + b2_ref[...], 0.0)
  gid = lax.broadcasted_iota(jnp.int32, (1, NUM_GRAPHS), 1)
  onehot = (batch_ref[...] == gid).astype(jnp.float32)
  cnt = jnp.sum(onehot, axis=0, keepdims=True)
  pooled = lax.dot_general(onehot, h2, (((0,), (0,)), ((), ())),
                           preferred_element_type=jnp.float32)
  pooled = pooled / jnp.maximum(cnt, 1.0).T
  out_ref[...] = jnp.dot(pooled, wlin_ref[...],
                         preferred_element_type=jnp.float32) + blin_ref[...]


_tc_table = pl.pallas_call(
    _tc_table_body,
    out_shape=jax.ShapeDtypeStruct((VOCAB, DIM), jnp.float32))

_tc_g1 = pl.pallas_call(
    _tc_g1_body,
    out_shape=jax.ShapeDtypeStruct((NP, DIM), jnp.float32))

_tc_g2 = pl.pallas_call(
    _tc_g2_body,
    out_shape=jax.ShapeDtypeStruct((NP, DIM), jnp.float32))

_tc_final = pl.pallas_call(
    _tc_final_body,
    out_shape=jax.ShapeDtypeStruct((NUM_GRAPHS, NUM_CLASSES), jnp.float32))


@jax.jit
def kernel(x, edge_index, batch, emb_table, W1, b1, W2, b2, Wlin, blin):
  x = x.astype(jnp.int32)
  pad_e = E_PAD - N_EDGES
  pad_n = NP - N_NODES
  pad_i = jnp.arange(pad_e, dtype=jnp.int32)
  src = jnp.concatenate([edge_index[0], N_NODES + pad_i % pad_n])
  dst = jnp.concatenate([edge_index[1], pad_i % NP]).reshape(NW, CPW, CH)
  dst_deg = jnp.concatenate(
      [edge_index[1], N_NODES + pad_i % pad_n]).reshape(NW, CPW, CH)
  xp = jnp.concatenate([x, jnp.zeros((pad_n,), jnp.int32)])
  batchp = jnp.concatenate(
      [batch, jnp.full((pad_n,), NUM_GRAPHS, jnp.int32)]).reshape(NP, 1)
  ones_c = jnp.ones((RPS,), jnp.float32)

  table_z = _tc_table(emb_table)
  deg0, deg1, h0 = _sc_deg_embed(dst_deg, ones_c, table_z, xp)
  deg0 = deg0.reshape(NP, 1)
  deg1 = deg1.reshape(NP, 1)
  g1 = _tc_g1(h0, deg0, deg1, W1)
  p1 = _sc_edge_agg(src, dst, g1)
  g2 = _tc_g2(p1, g1, deg0, deg1, b1.reshape(1, DIM), W2)
  p2 = _sc_edge_agg(src, dst, g2)
  return _tc_final(p2, g2, deg0, deg1, b2.reshape(1, DIM), batchp, Wlin,
                   blin.reshape(1, NUM_CLASSES))


# speedup vs baseline: 23.0126x; 1.0085x over previous
import functools

import jax
import jax.numpy as jnp
from jax import lax
from jax.experimental import pallas as pl
from jax.experimental.pallas import tpu as pltpu
from jax.experimental.pallas import tpu_sc as plsc

N_NODES = 10000
N_EDGES = 320000
VOCAB = 1000
DIM = 128
NUM_CLASSES = 10
NUM_GRAPHS = 64

NC, NS = 2, 16
NW = NC * NS
CH = 128
CPW = 80
E_PAD = NW * CPW * CH
NP = 10240
N_NCH = NP // CH
RPS = NP // NS

_mesh = plsc.VectorSubcoreMesh(
    core_axis_name="c", subcore_axis_name="s", num_cores=NC, num_subcores=NS)


def _wid():
  return lax.axis_index("s") * NC + lax.axis_index("c")


@functools.partial(
    pl.kernel,
    out_type=(
        jax.ShapeDtypeStruct((NP,), jnp.float32),
        jax.ShapeDtypeStruct((NP,), jnp.float32),
        jax.ShapeDtypeStruct((NP, DIM), jnp.float32),
    ),
    mesh=_mesh,
    scratch_types=(
        pltpu.VMEM((CPW, CH), jnp.int32),
        pltpu.VMEM((CH,), jnp.float32),
        pltpu.VMEM((CH,), jnp.int32),
        pltpu.VMEM((CH, DIM), jnp.float32),
        pltpu.SemaphoreType.DMA,
        pltpu.VMEM_SHARED((NP,), jnp.float32),
    ),
)
def _sc_deg_embed(dst_hbm, ones_hbm, table_hbm, x_hbm, deg0_out, deg1_out,
                  h0_out, dst_v, ones_v, x_v, rows_v, dsem, accd):
  c = lax.axis_index("c")
  s = lax.axis_index("s")
  w = _wid()

  pltpu.sync_copy(ones_hbm.at[pl.ds(0, RPS)], accd.at[pl.ds(s * RPS, RPS)])
  pltpu.sync_copy(ones_hbm.at[pl.ds(0, CH)], ones_v)
  pltpu.sync_copy(dst_hbm.at[w], dst_v)
  plsc.subcore_barrier()

  @pl.loop(0, CPW)
  def _count(j):
    pltpu.async_copy(ones_v, accd.at[dst_v.at[j]], dsem, add=True)

  @pl.loop(w, N_NCH, step=NW)
  def _embed(t):
    pltpu.sync_copy(x_hbm.at[pl.ds(t * CH, CH)], x_v)
    pltpu.sync_copy(table_hbm.at[x_v], rows_v)
    pltpu.sync_copy(rows_v, h0_out.at[pl.ds(t * CH, CH)])

  @pl.loop(0, CPW)
  def _drain(j):
    pltpu.make_async_copy(ones_v, accd.at[pl.ds(0, CH)], dsem).wait()

  plsc.subcore_barrier()

  @pl.when(c == 0)
  def _():
    pltpu.sync_copy(accd.at[pl.ds(s * RPS, RPS)],
                    deg0_out.at[pl.ds(s * RPS, RPS)])

  @pl.when(c == 1)
  def _():
    pltpu.sync_copy(accd.at[pl.ds(s * RPS, RPS)],
                    deg1_out.at[pl.ds(s * RPS, RPS)])


NBUF = 2
NH = 2
HCPW = CPW // NH
HGRP = HCPW // NBUF


@functools.partial(
    pl.kernel,
    out_type=jax.ShapeDtypeStruct((NC, NP, DIM), jnp.float32),
    mesh=_mesh,
    scratch_types=(
        pltpu.VMEM((HCPW * CH,), jnp.int32),
        pltpu.VMEM((HCPW, CH), jnp.int32),
        tuple(pltpu.VMEM((CH, DIM), jnp.float32) for _ in range(NBUF)),
        tuple(pltpu.SemaphoreType.DMA for _ in range(NBUF)),
        tuple(pltpu.SemaphoreType.DMA for _ in range(NBUF)),
        pltpu.VMEM_SHARED((NP, DIM), jnp.float32),
    ),
)
def _sc_edge_agg(src_hbm, dst_hbm, g_hbm, acc_out, src_v, dst_v, rows,
                 gsem, ssem, acc):
  c = lax.axis_index("c")
  s = lax.axis_index("s")
  w = _wid()

  pltpu.sync_copy(g_hbm.at[pl.ds(s * RPS, RPS)], acc.at[pl.ds(s * RPS, RPS)])
  plsc.subcore_barrier()

  def gather(j, b):
    pltpu.async_copy(g_hbm.at[src_v.at[pl.ds(j * CH, CH)]], rows[b], gsem[b])

  def gather_wait(b):
    pltpu.make_async_copy(g_hbm.at[pl.ds(0, CH)], rows[b], gsem[b]).wait()

  def scatter(j, b):
    pltpu.async_copy(rows[b], acc.at[dst_v.at[j]], ssem[b], add=True)

  def scatter_wait(b):
    pltpu.make_async_copy(rows[b], acc.at[pl.ds(0, CH)], ssem[b]).wait()

  for h in range(NH):
    pltpu.sync_copy(
        src_hbm.at[pl.ds((w * CPW + h * HCPW) * CH, HCPW * CH)], src_v)
    pltpu.sync_copy(dst_hbm.at[w].at[pl.ds(h * HCPW, HCPW)], dst_v)

    for b in range(NBUF):
      gather(b, b)

    @pl.loop(0, HGRP - 1)
    def _agg(i):
      base = i * NBUF
      for b in range(NBUF):
        gather_wait(b)
        scatter(base + b, b)
      for b in range(NBUF):
        scatter_wait(b)
        gather(base + NBUF + b, b)

    for b in range(NBUF):
      gather_wait(b)
      scatter(HCPW - NBUF + b, b)
    for b in range(NBUF):
      scatter_wait(b)

  plsc.subcore_barrier()
  pltpu.sync_copy(acc.at[pl.ds(s * RPS, RPS)],
                  acc_out.at[c].at[pl.ds(s * RPS, RPS)])


def _dinv(deg0_ref, deg1_ref):
  deg = deg0_ref[...] + deg1_ref[...] - 1.0
  return lax.rsqrt(deg)


def _rowmask():
  return (lax.broadcasted_iota(jnp.int32, (NP, 1), 0) < N_NODES).astype(
      jnp.float32)


def _tc_g1_body(h0_ref, x_ref, deg0_ref, deg1_ref, w1_ref, g1_ref):
  d = _dinv(deg0_ref, deg1_ref) * (x_ref[...] != 0).astype(jnp.float32)
  g1_ref[...] = d * jnp.dot(h0_ref[...], w1_ref[...],
                            preferred_element_type=jnp.float32)


def _tc_g2_body(p_ref, g1_ref, deg0_ref, deg1_ref, b1_ref, w2_ref, g2_ref):
  d = _dinv(deg0_ref, deg1_ref)
  p = p_ref[...]
  p_raw = p[0] + p[1] - g1_ref[...]
  h1 = jnp.maximum(d * p_raw + b1_ref[...], 0.0)
  g2_ref[...] = (d * _rowmask()) * jnp.dot(h1, w2_ref[...],
                                           preferred_element_type=jnp.float32)


def _tc_final_body(q_ref, g2_ref, deg0_ref, deg1_ref, b2_ref, batch_ref,
                   wlin_ref, blin_ref, out_ref):
  d = _dinv(deg0_ref, deg1_ref)
  q = q_ref[...]
  p_raw = q[0] + q[1] - g2_ref[...]
  h2 = jnp.maximum(d * p_raw + b2_ref[...], 0.0)
  gid = lax.broadcasted_iota(jnp.int32, (1, NUM_GRAPHS), 1)
  onehot = (batch_ref[...] == gid).astype(jnp.float32)
  cnt = jnp.sum(onehot, axis=0, keepdims=True)
  pooled = lax.dot_general(onehot, h2, (((0,), (0,)), ((), ())),
                           preferred_element_type=jnp.float32)
  pooled = pooled / jnp.maximum(cnt, 1.0).T
  out_ref[...] = jnp.dot(pooled, wlin_ref[...],
                         preferred_element_type=jnp.float32) + blin_ref[...]


_tc_g1 = pl.pallas_call(
    _tc_g1_body,
    out_shape=jax.ShapeDtypeStruct((NP, DIM), jnp.float32))

_tc_g2 = pl.pallas_call(
    _tc_g2_body,
    out_shape=jax.ShapeDtypeStruct((NP, DIM), jnp.float32))

_tc_final = pl.pallas_call(
    _tc_final_body,
    out_shape=jax.ShapeDtypeStruct((NUM_GRAPHS, NUM_CLASSES), jnp.float32))


@jax.jit
def kernel(x, edge_index, batch, emb_table, W1, b1, W2, b2, Wlin, blin):
  x = x.astype(jnp.int32)
  pad_e = E_PAD - N_EDGES
  pad_n = NP - N_NODES
  pad_i = jnp.arange(pad_e, dtype=jnp.int32)
  src = jnp.concatenate([edge_index[0], N_NODES + pad_i % pad_n])
  dst = jnp.concatenate([edge_index[1], pad_i % NP]).reshape(NW, CPW, CH)
  dst_deg = jnp.concatenate(
      [edge_index[1], N_NODES + pad_i % pad_n]).reshape(NW, CPW, CH)
  xp = jnp.concatenate([x, jnp.zeros((pad_n,), jnp.int32)])
  batchp = jnp.concatenate(
      [batch, jnp.full((pad_n,), NUM_GRAPHS, jnp.int32)]).reshape(NP, 1)
  ones_c = jnp.ones((RPS,), jnp.float32)

  deg0, deg1, h0 = _sc_deg_embed(dst_deg, ones_c, emb_table, xp)
  deg0 = deg0.reshape(NP, 1)
  deg1 = deg1.reshape(NP, 1)
  g1 = _tc_g1(h0, xp.reshape(NP, 1), deg0, deg1, W1)
  p1 = _sc_edge_agg(src, dst, g1)
  g2 = _tc_g2(p1, g1, deg0, deg1, b1.reshape(1, DIM), W2)
  p2 = _sc_edge_agg(src, dst, g2)
  return _tc_final(p2, g2, deg0, deg1, b2.reshape(1, DIM), batchp, Wlin,
                   blin.reshape(1, NUM_CLASSES))


# speedup vs baseline: 24.6500x; 1.0712x over previous
import functools

import jax
import jax.numpy as jnp
from jax import lax
from jax.experimental import pallas as pl
from jax.experimental.pallas import tpu as pltpu
from jax.experimental.pallas import tpu_sc as plsc

N = 10000
N_EDGES = 320000
VOCAB = 1000
DIM = 128
NUM_CLASSES = 10
NUM_GRAPHS = 64

NC, NS = 2, 16
NW = NC * NS
CH = 128
NCHUNK = N_EDGES // CH
NGFULL = 312
MAXG = 10
MAXCH = 80
RPS = 640
RLAST = N - 15 * RPS
NPD = 10240
NECH = 78

_mesh = plsc.VectorSubcoreMesh(
    core_axis_name="c", subcore_axis_name="s", num_cores=NC, num_subcores=NS)


def _wid():
  return lax.axis_index("s") * NC + lax.axis_index("c")


def _nchunks(w):
  return jnp.where(w < 24, 80, jnp.where(w < 28, 72, 73))


def _load_all_groups(edge3, which, buf, sem, w):
  for gi in range(MAXG):
    @pl.when(w + NW * gi < NGFULL)
    def _():
      pltpu.async_copy(
          edge3.at[which].at[pl.ds((w + NW * gi) * 8, 8)],
          buf.at[pl.ds(gi * 8, 8)], sem)
  @pl.when(w >= 28)
  def _():
    pltpu.async_copy(edge3.at[which].at[pl.ds(NGFULL * 8, 4)],
                     buf.at[pl.ds(72, 4)], sem)


def _drain_all_groups(edge3, buf, sem, w):
  ng = jnp.where(w < 24, MAXG, MAXG - 1)
  @pl.loop(0, ng)
  def _(i):
    pltpu.make_async_copy(edge3.at[0].at[pl.ds(0, 8)],
                          buf.at[pl.ds(0, 8)], sem).wait()
  @pl.when(w >= 28)
  def _():
    pltpu.make_async_copy(edge3.at[0].at[pl.ds(0, 4)],
                          buf.at[pl.ds(0, 4)], sem).wait()


def _load_pass_groups(edge3, which, buf, sem, w, p):
  for gl in range(5):
    gi = p * 5 + gl
    @pl.when(w + NW * gi < NGFULL)
    def _():
      pltpu.async_copy(
          edge3.at[which].at[pl.ds((w + NW * gi) * 8, 8)],
          buf.at[pl.ds(gl * 8, 8)], sem)
  if p == 1:
    @pl.when(w >= 28)
    def _():
      pltpu.async_copy(edge3.at[which].at[pl.ds(NGFULL * 8, 4)],
                       buf.at[pl.ds(32, 4)], sem)


def _drain_pass_groups(edge3, buf, sem, w, p, narrays):
  if p == 0:
    n = 5 * narrays
  else:
    n = narrays * jnp.where(w < 24, 5, 4)
  @pl.loop(0, n)
  def _(i):
    pltpu.make_async_copy(edge3.at[0].at[pl.ds(0, 8)],
                          buf.at[pl.ds(0, 8)], sem).wait()
  if p == 1:
    @pl.when(w >= 28)
    def _():
      for _k in range(narrays):
        pltpu.make_async_copy(edge3.at[0].at[pl.ds(0, 4)],
                              buf.at[pl.ds(0, 4)], sem).wait()


def _init_rows(dst_ref, src_full, src_last, s):
  @pl.when(s < 15)
  def _():
    pltpu.sync_copy(src_full, dst_ref.at[pl.ds(s * RPS, RPS)])
  @pl.when(s == 15)
  def _():
    pltpu.sync_copy(src_last, dst_ref.at[pl.ds(15 * RPS, RLAST)])


def _writeout_rows(src_ref, out_ref, s):
  @pl.when(s < 15)
  def _():
    pltpu.sync_copy(src_ref.at[pl.ds(s * RPS, RPS)],
                    out_ref.at[pl.ds(s * RPS, RPS)])
  @pl.when(s == 15)
  def _():
    pltpu.sync_copy(src_ref.at[pl.ds(15 * RPS, RLAST)],
                    out_ref.at[pl.ds(15 * RPS, RLAST)])


@functools.partial(
    pl.kernel,
    out_type=(
        jax.ShapeDtypeStruct((NPD,), jnp.float32),
        jax.ShapeDtypeStruct((NPD,), jnp.float32),
        jax.ShapeDtypeStruct((N, DIM), jnp.float32),
    ),
    mesh=_mesh,
    scratch_types=(
        pltpu.VMEM((MAXCH, CH), jnp.int32),
        pltpu.VMEM((CH,), jnp.float32),
        pltpu.VMEM((CH,), jnp.int32),
        pltpu.VMEM((CH, DIM), jnp.float32),
        pltpu.SemaphoreType.DMA,
        pltpu.SemaphoreType.DMA,
        pltpu.VMEM_SHARED((NPD,), jnp.float32),
    ),
)
def _sc_deg_embed(edge3, ones_hbm, zeros1_hbm, table_hbm, x_hbm,
                  deg0_out, deg1_out, h0_out,
                  dst_v, ones_v, x_v, rows_v, lsem, dsem, accd):
  c = lax.axis_index("c")
  s = lax.axis_index("s")
  w = _wid()

  _load_all_groups(edge3, 1, dst_v, lsem, w)
  @pl.when(c == 0)
  def _():
    pltpu.sync_copy(ones_hbm, accd.at[pl.ds(s * RPS, RPS)])
  @pl.when(c == 1)
  def _():
    pltpu.sync_copy(zeros1_hbm, accd.at[pl.ds(s * RPS, RPS)])
  pltpu.sync_copy(ones_hbm.at[pl.ds(0, CH)], ones_v)
  _drain_all_groups(edge3, dst_v, lsem, w)
  plsc.subcore_barrier()

  nch = _nchunks(w)

  @pl.loop(0, nch)
  def _count(j):
    row = j + jnp.where(j == 72, w - 28, 0)
    pltpu.async_copy(ones_v, accd.at[dst_v.at[row]], dsem, add=True)

  @pl.loop(w, NECH, step=NW)
  def _embed(t):
    pltpu.sync_copy(x_hbm.at[pl.ds(t * CH, CH)], x_v)
    pltpu.sync_copy(table_hbm.at[x_v], rows_v)
    pltpu.sync_copy(rows_v, h0_out.at[pl.ds(t * CH, CH)])

  @pl.when(w == 31)
  def _():
    pltpu.sync_copy(x_hbm.at[pl.ds(N - CH, CH)], x_v)
    pltpu.sync_copy(table_hbm.at[x_v.at[pl.ds(CH - 16, 16)]],
                    rows_v.at[pl.ds(0, 16)])
    pltpu.sync_copy(rows_v.at[pl.ds(0, 16)],
                    h0_out.at[pl.ds(NECH * CH, 16)])

  @pl.loop(0, nch)
  def _drain(j):
    pltpu.make_async_copy(ones_v, accd.at[pl.ds(0, CH)], dsem).wait()

  plsc.subcore_barrier()

  @pl.when(c == 0)
  def _():
    pltpu.sync_copy(accd.at[pl.ds(s * RPS, RPS)],
                    deg0_out.at[pl.ds(s * RPS, RPS)])

  @pl.when(c == 1)
  def _():
    pltpu.sync_copy(accd.at[pl.ds(s * RPS, RPS)],
                    deg1_out.at[pl.ds(s * RPS, RPS)])


NBUF = 2


@functools.partial(
    pl.kernel,
    out_type=jax.ShapeDtypeStruct((NC, N, DIM), jnp.float32),
    mesh=_mesh,
    scratch_types=(
        pltpu.VMEM((40, CH), jnp.int32),
        pltpu.VMEM((40, CH), jnp.int32),
        tuple(pltpu.VMEM((CH, DIM), jnp.float32) for _ in range(NBUF)),
        pltpu.SemaphoreType.DMA,
        tuple(pltpu.SemaphoreType.DMA for _ in range(NBUF)),
        tuple(pltpu.SemaphoreType.DMA for _ in range(NBUF)),
        pltpu.VMEM_SHARED((N, DIM), jnp.float32),
    ),
)
def _sc_edge_agg(edge3, zeros_hbm, g_hbm, acc_out, src_v, dst_v, rows,
                 lsem, gsem, ssem, acc):
  c = lax.axis_index("c")
  s = lax.axis_index("s")
  w = _wid()

  _load_pass_groups(edge3, 0, src_v, lsem, w, 0)
  _load_pass_groups(edge3, 1, dst_v, lsem, w, 0)
  @pl.when(c == 0)
  def _():
    _init_rows(acc, g_hbm.at[pl.ds(s * RPS, RPS)],
               g_hbm.at[pl.ds(15 * RPS, RLAST)], s)
  @pl.when(c == 1)
  def _():
    _init_rows(acc, zeros_hbm, zeros_hbm.at[pl.ds(0, RLAST)], s)
  _drain_pass_groups(edge3, dst_v, lsem, w, 0, 2)
  plsc.subcore_barrier()

  def gather(j, b):
    pltpu.async_copy(g_hbm.at[src_v.at[j]], rows[b], gsem[b])

  def gather_wait(b):
    pltpu.make_async_copy(g_hbm.at[pl.ds(0, CH)], rows[b], gsem[b]).wait()

  def scatter(j, b):
    pltpu.async_copy(rows[b], acc.at[dst_v.at[j]], ssem[b], add=True)

  def scatter_wait(b):
    pltpu.make_async_copy(rows[b], acc.at[pl.ds(0, CH)], ssem[b]).wait()

  def run_pipeline(npairs):
    for b in range(NBUF):
      gather(b, b)

    @pl.loop(0, npairs - 1)
    def _agg(i):
      base = i * NBUF
      for b in range(NBUF):
        gather_wait(b)
        scatter(base + b, b)
      for b in range(NBUF):
        scatter_wait(b)
        gather(base + NBUF + b, b)

    for b in range(NBUF):
      gather_wait(b)
      scatter((npairs - 1) * NBUF + b, b)
    for b in range(NBUF):
      scatter_wait(b)

  run_pipeline(20)

  _load_pass_groups(edge3, 0, src_v, lsem, w, 1)
  _load_pass_groups(edge3, 1, dst_v, lsem, w, 1)
  _drain_pass_groups(edge3, dst_v, lsem, w, 1, 2)
  run_pipeline(jnp.where(w < 24, 20, 16))

  @pl.when(w >= 28)
  def _():
    row = 32 + (w - 28)
    gather(row, 0)
    gather_wait(0)
    scatter(row, 0)
    scatter_wait(0)

  plsc.subcore_barrier()
  _writeout_rows(acc, acc_out.at[c], s)


def _dinv(deg0_ref, deg1_ref):
  return lax.rsqrt(deg0_ref[...][:N] + deg1_ref[...][:N])


def _tc_g1_body(h0_ref, x_ref, deg0_ref, deg1_ref, w1_ref, g1_ref):
  d = _dinv(deg0_ref, deg1_ref) * (x_ref[...] != 0).astype(jnp.float32)
  g1_ref[...] = d * jnp.dot(h0_ref[...], w1_ref[...],
                            preferred_element_type=jnp.float32)


def _tc_g2_body(p_ref, deg0_ref, deg1_ref, b1_ref, w2_ref, g2_ref):
  d = _dinv(deg0_ref, deg1_ref)
  p = p_ref[...]
  h1 = jnp.maximum(d * (p[0] + p[1]) + b1_ref[...], 0.0)
  g2_ref[...] = d * jnp.dot(h1, w2_ref[...],
                            preferred_element_type=jnp.float32)


def _tc_final_body(q_ref, deg0_ref, deg1_ref, b2_ref, batch_ref,
                   wlin_ref, blin_ref, out_ref):
  d = _dinv(deg0_ref, deg1_ref)
  q = q_ref[...]
  h2 = jnp.maximum(d * (q[0] + q[1]) + b2_ref[...], 0.0)
  gid = lax.broadcasted_iota(jnp.int32, (1, NUM_GRAPHS), 1)
  onehot = (batch_ref[...] == gid).astype(jnp.float32)
  cnt = jnp.sum(onehot, axis=0, keepdims=True)
  pooled = lax.dot_general(onehot, h2, (((0,), (0,)), ((), ())),
                           preferred_element_type=jnp.float32)
  pooled = pooled / jnp.maximum(cnt, 1.0).T
  out_ref[...] = jnp.dot(pooled, wlin_ref[...],
                         preferred_element_type=jnp.float32) + blin_ref[...]


_tc_g1 = pl.pallas_call(
    _tc_g1_body,
    out_shape=jax.ShapeDtypeStruct((N, DIM), jnp.float32))

_tc_g2 = pl.pallas_call(
    _tc_g2_body,
    out_shape=jax.ShapeDtypeStruct((N, DIM), jnp.float32))

_tc_final = pl.pallas_call(
    _tc_final_body,
    out_shape=jax.ShapeDtypeStruct((NUM_GRAPHS, NUM_CLASSES), jnp.float32))


@jax.jit
def kernel(x, edge_index, batch, emb_table, W1, b1, W2, b2, Wlin, blin):
  x = x.astype(jnp.int32)
  edge3 = edge_index.reshape(2, NCHUNK, CH)
  ones_c = jnp.ones((RPS,), jnp.float32)
  zeros1 = jnp.zeros((RPS,), jnp.float32)
  zrows = jnp.zeros((RPS, DIM), jnp.float32)

  deg0, deg1, h0 = _sc_deg_embed(edge3, ones_c, zeros1, emb_table, x)
  deg0 = deg0.reshape(NPD, 1)
  deg1 = deg1.reshape(NPD, 1)
  g1 = _tc_g1(h0, x.reshape(N, 1), deg0, deg1, W1)
  p1 = _sc_edge_agg(edge3, zrows, g1)
  g2 = _tc_g2(p1, deg0, deg1, b1.reshape(1, DIM), W2)
  p2 = _sc_edge_agg(edge3, zrows, g2)
  return _tc_final(p2, deg0, deg1, b2.reshape(1, DIM), batch.reshape(N, 1),
                   Wlin, blin.reshape(1, NUM_CLASSES))


# speedup vs baseline: 24.7721x; 1.0050x over previous
import functools

import jax
import jax.numpy as jnp
from jax import lax
from jax.experimental import pallas as pl
from jax.experimental.pallas import tpu as pltpu
from jax.experimental.pallas import tpu_sc as plsc

N = 10000
N_EDGES = 320000
VOCAB = 1000
DIM = 128
NUM_CLASSES = 10
NUM_GRAPHS = 64

NC, NS = 2, 16
NW = NC * NS
CH = 128
NCHUNK = N_EDGES // CH
NGFULL = 312
MAXG = 10
MAXCH = 80
RPS = 640
RLAST = N - 15 * RPS
NPD = 10240
NECH = 78

_mesh = plsc.VectorSubcoreMesh(
    core_axis_name="c", subcore_axis_name="s", num_cores=NC, num_subcores=NS)


def _wid():
  return lax.axis_index("s") * NC + lax.axis_index("c")


def _nchunks(w):
  return jnp.where(w < 24, 80, jnp.where(w < 28, 72, 73))


def _load_all_groups(edge3, which, buf, sem, w):
  for gi in range(MAXG):
    @pl.when(w + NW * gi < NGFULL)
    def _():
      pltpu.async_copy(
          edge3.at[which].at[pl.ds((w + NW * gi) * 8, 8)],
          buf.at[pl.ds(gi * 8, 8)], sem)
  @pl.when(w >= 28)
  def _():
    pltpu.async_copy(edge3.at[which].at[pl.ds(NGFULL * 8, 4)],
                     buf.at[pl.ds(72, 4)], sem)


def _drain_all_groups(edge3, buf, sem, w):
  ng = jnp.where(w < 24, MAXG, MAXG - 1)
  @pl.loop(0, ng)
  def _(i):
    pltpu.make_async_copy(edge3.at[0].at[pl.ds(0, 8)],
                          buf.at[pl.ds(0, 8)], sem).wait()
  @pl.when(w >= 28)
  def _():
    pltpu.make_async_copy(edge3.at[0].at[pl.ds(0, 4)],
                          buf.at[pl.ds(0, 4)], sem).wait()


def _load_pass_groups(edge3, which, buf, sem, w, p):
  for gl in range(5):
    gi = p * 5 + gl
    @pl.when(w + NW * gi < NGFULL)
    def _():
      pltpu.async_copy(
          edge3.at[which].at[pl.ds((w + NW * gi) * 8, 8)],
          buf.at[pl.ds(gl * 8, 8)], sem)
  if p == 1:
    @pl.when(w >= 28)
    def _():
      pltpu.async_copy(edge3.at[which].at[pl.ds(NGFULL * 8, 4)],
                       buf.at[pl.ds(32, 4)], sem)


def _drain_pass_groups(edge3, buf, sem, w, p, narrays):
  if p == 0:
    n = 5 * narrays
  else:
    n = narrays * jnp.where(w < 24, 5, 4)
  @pl.loop(0, n)
  def _(i):
    pltpu.make_async_copy(edge3.at[0].at[pl.ds(0, 8)],
                          buf.at[pl.ds(0, 8)], sem).wait()
  if p == 1:
    @pl.when(w >= 28)
    def _():
      for _k in range(narrays):
        pltpu.make_async_copy(edge3.at[0].at[pl.ds(0, 4)],
                              buf.at[pl.ds(0, 4)], sem).wait()


def _init_rows(dst_ref, src_full, src_last, s):
  @pl.when(s < 15)
  def _():
    pltpu.sync_copy(src_full, dst_ref.at[pl.ds(s * RPS, RPS)])
  @pl.when(s == 15)
  def _():
    pltpu.sync_copy(src_last, dst_ref.at[pl.ds(15 * RPS, RLAST)])


def _writeout_rows(src_ref, out_ref, s):
  @pl.when(s < 15)
  def _():
    pltpu.sync_copy(src_ref.at[pl.ds(s * RPS, RPS)],
                    out_ref.at[pl.ds(s * RPS, RPS)])
  @pl.when(s == 15)
  def _():
    pltpu.sync_copy(src_ref.at[pl.ds(15 * RPS, RLAST)],
                    out_ref.at[pl.ds(15 * RPS, RLAST)])


@functools.partial(
    pl.kernel,
    out_type=(
        jax.ShapeDtypeStruct((NPD,), jnp.float32),
        jax.ShapeDtypeStruct((NPD,), jnp.float32),
        jax.ShapeDtypeStruct((N, DIM), jnp.float32),
    ),
    mesh=_mesh,
    scratch_types=(
        pltpu.VMEM((MAXCH, CH), jnp.int32),
        pltpu.VMEM((CH,), jnp.float32),
        pltpu.VMEM((CH,), jnp.int32),
        pltpu.VMEM((CH, DIM), jnp.float32),
        pltpu.SemaphoreType.DMA,
        pltpu.SemaphoreType.DMA,
        pltpu.VMEM_SHARED((NPD,), jnp.float32),
    ),
)
def _sc_deg_embed(edge3, ones_hbm, zeros1_hbm, table_hbm, x_hbm,
                  deg0_out, deg1_out, h0_out,
                  dst_v, ones_v, x_v, rows_v, lsem, dsem, accd):
  c = lax.axis_index("c")
  s = lax.axis_index("s")
  w = _wid()

  _load_all_groups(edge3, 1, dst_v, lsem, w)
  @pl.when(c == 0)
  def _():
    pltpu.sync_copy(ones_hbm, accd.at[pl.ds(s * RPS, RPS)])
  @pl.when(c == 1)
  def _():
    pltpu.sync_copy(zeros1_hbm, accd.at[pl.ds(s * RPS, RPS)])
  pltpu.sync_copy(ones_hbm.at[pl.ds(0, CH)], ones_v)
  _drain_all_groups(edge3, dst_v, lsem, w)
  plsc.subcore_barrier()

  nch = _nchunks(w)

  @pl.loop(0, nch)
  def _count(j):
    row = j + jnp.where((j == 72) & (w >= 28), w - 28, 0)
    pltpu.async_copy(ones_v, accd.at[dst_v.at[row]], dsem, add=True)

  @pl.loop(w, NECH, step=NW)
  def _embed(t):
    pltpu.sync_copy(x_hbm.at[pl.ds(t * CH, CH)], x_v)
    pltpu.sync_copy(table_hbm.at[x_v], rows_v)
    pltpu.sync_copy(rows_v, h0_out.at[pl.ds(t * CH, CH)])

  @pl.when(w == 31)
  def _():
    pltpu.sync_copy(x_hbm.at[pl.ds(N - CH, CH)], x_v)
    pltpu.sync_copy(table_hbm.at[x_v.at[pl.ds(CH - 16, 16)]],
                    rows_v.at[pl.ds(0, 16)])
    pltpu.sync_copy(rows_v.at[pl.ds(0, 16)],
                    h0_out.at[pl.ds(NECH * CH, 16)])

  @pl.loop(0, nch)
  def _drain(j):
    pltpu.make_async_copy(ones_v, accd.at[pl.ds(0, CH)], dsem).wait()

  plsc.subcore_barrier()

  @pl.when(c == 0)
  def _():
    pltpu.sync_copy(accd.at[pl.ds(s * RPS, RPS)],
                    deg0_out.at[pl.ds(s * RPS, RPS)])

  @pl.when(c == 1)
  def _():
    pltpu.sync_copy(accd.at[pl.ds(s * RPS, RPS)],
                    deg1_out.at[pl.ds(s * RPS, RPS)])


NBUF = 2


@functools.partial(
    pl.kernel,
    out_type=jax.ShapeDtypeStruct((NC, N, DIM), jnp.float32),
    mesh=_mesh,
    scratch_types=(
        pltpu.VMEM((40, CH), jnp.int32),
        pltpu.VMEM((40, CH), jnp.int32),
        tuple(pltpu.VMEM((CH, DIM), jnp.float32) for _ in range(NBUF)),
        pltpu.SemaphoreType.DMA,
        tuple(pltpu.SemaphoreType.DMA for _ in range(NBUF)),
        tuple(pltpu.SemaphoreType.DMA for _ in range(NBUF)),
        pltpu.VMEM_SHARED((N, DIM), jnp.float32),
    ),
)
def _sc_edge_agg(edge3, zeros_hbm, g_hbm, acc_out, src_v, dst_v, rows,
                 lsem, gsem, ssem, acc):
  c = lax.axis_index("c")
  s = lax.axis_index("s")
  w = _wid()

  _load_pass_groups(edge3, 0, src_v, lsem, w, 0)
  _load_pass_groups(edge3, 1, dst_v, lsem, w, 0)
  @pl.when(c == 0)
  def _():
    _init_rows(acc, g_hbm.at[pl.ds(s * RPS, RPS)],
               g_hbm.at[pl.ds(15 * RPS, RLAST)], s)
  @pl.when(c == 1)
  def _():
    _init_rows(acc, zeros_hbm, zeros_hbm.at[pl.ds(0, RLAST)], s)
  _drain_pass_groups(edge3, dst_v, lsem, w, 0, 2)
  plsc.subcore_barrier()

  def gather(j, b):
    pltpu.async_copy(g_hbm.at[src_v.at[j]], rows[b], gsem[b])

  def gather_wait(b):
    pltpu.make_async_copy(g_hbm.at[pl.ds(0, CH)], rows[b], gsem[b]).wait()

  def scatter(j, b):
    pltpu.async_copy(rows[b], acc.at[dst_v.at[j]], ssem[b], add=True)

  def scatter_wait(b):
    pltpu.make_async_copy(rows[b], acc.at[pl.ds(0, CH)], ssem[b]).wait()

  def run_pipeline(npairs):
    for b in range(NBUF):
      gather(b, b)

    @pl.loop(0, npairs - 1)
    def _agg(i):
      base = i * NBUF
      for b in range(NBUF):
        gather_wait(b)
        scatter(base + b, b)
      for b in range(NBUF):
        scatter_wait(b)
        gather(base + NBUF + b, b)

    for b in range(NBUF):
      gather_wait(b)
      scatter((npairs - 1) * NBUF + b, b)
    for b in range(NBUF):
      scatter_wait(b)

  run_pipeline(20)

  _load_pass_groups(edge3, 0, src_v, lsem, w, 1)
  _load_pass_groups(edge3, 1, dst_v, lsem, w, 1)
  _drain_pass_groups(edge3, dst_v, lsem, w, 1, 2)
  run_pipeline(jnp.where(w < 24, 20, 16))

  @pl.when(w >= 28)
  def _():
    row = 32 + (w - 28)
    gather(row, 0)
    gather_wait(0)
    scatter(row, 0)
    scatter_wait(0)

  plsc.subcore_barrier()
  _writeout_rows(acc, acc_out.at[c], s)


def _dinv(deg0_ref, deg1_ref):
  return lax.rsqrt(deg0_ref[...][:N] + deg1_ref[...][:N])


def _tc_g1_body(h0_ref, x_ref, deg0_ref, deg1_ref, w1_ref, g1_ref):
  d = _dinv(deg0_ref, deg1_ref) * (x_ref[...] != 0).astype(jnp.float32)
  g1_ref[...] = d * jnp.dot(h0_ref[...], w1_ref[...],
                            preferred_element_type=jnp.float32)


def _tc_g2_body(p_ref, deg0_ref, deg1_ref, b1_ref, w2_ref, g2_ref):
  d = _dinv(deg0_ref, deg1_ref)
  p = p_ref[...]
  h1 = jnp.maximum(d * (p[0] + p[1]) + b1_ref[...], 0.0)
  g2_ref[...] = d * jnp.dot(h1, w2_ref[...],
                            preferred_element_type=jnp.float32)


def _tc_final_body(q_ref, deg0_ref, deg1_ref, b2_ref, batch_ref,
                   wlin_ref, blin_ref, out_ref):
  d = _dinv(deg0_ref, deg1_ref)
  q = q_ref[...]
  h2 = jnp.maximum(d * (q[0] + q[1]) + b2_ref[...], 0.0)
  gid = lax.broadcasted_iota(jnp.int32, (1, NUM_GRAPHS), 1)
  onehot = (batch_ref[...] == gid).astype(jnp.float32)
  cnt = jnp.sum(onehot, axis=0, keepdims=True)
  pooled = lax.dot_general(onehot, h2, (((0,), (0,)), ((), ())),
                           preferred_element_type=jnp.float32)
  pooled = pooled / jnp.maximum(cnt, 1.0).T
  out_ref[...] = jnp.dot(pooled, wlin_ref[...],
                         preferred_element_type=jnp.float32) + blin_ref[...]


_tc_g1 = pl.pallas_call(
    _tc_g1_body,
    out_shape=jax.ShapeDtypeStruct((N, DIM), jnp.float32))

_tc_g2 = pl.pallas_call(
    _tc_g2_body,
    out_shape=jax.ShapeDtypeStruct((N, DIM), jnp.float32))

_tc_final = pl.pallas_call(
    _tc_final_body,
    out_shape=jax.ShapeDtypeStruct((NUM_GRAPHS, NUM_CLASSES), jnp.float32))


@jax.jit
def kernel(x, edge_index, batch, emb_table, W1, b1, W2, b2, Wlin, blin):
  x = x.astype(jnp.int32)
  edge3 = edge_index.reshape(2, NCHUNK, CH)
  ones_c = jnp.ones((RPS,), jnp.float32)
  zeros1 = jnp.zeros((RPS,), jnp.float32)
  zrows = jnp.zeros((RPS, DIM), jnp.float32)

  deg0, deg1, h0 = _sc_deg_embed(edge3, ones_c, zeros1, emb_table, x)
  deg0 = deg0.reshape(NPD, 1)
  deg1 = deg1.reshape(NPD, 1)
  g1 = _tc_g1(h0, x.reshape(N, 1), deg0, deg1, W1)
  p1 = _sc_edge_agg(edge3, zrows, g1)
  g2 = _tc_g2(p1, deg0, deg1, b1.reshape(1, DIM), W2)
  p2 = _sc_edge_agg(edge3, zrows, g2)
  return _tc_final(p2, deg0, deg1, b2.reshape(1, DIM), batch.reshape(N, 1),
                   Wlin, blin.reshape(1, NUM_CLASSES))


# speedup vs baseline: 26.8488x; 1.0838x over previous
import functools

import jax
import jax.numpy as jnp
from jax import lax
from jax.experimental import pallas as pl
from jax.experimental.pallas import tpu as pltpu
from jax.experimental.pallas import tpu_sc as plsc

N = 10000
N_EDGES = 320000
VOCAB = 1000
DIM = 128
NUM_CLASSES = 10
NUM_GRAPHS = 64

NC, NS = 2, 16
NW = NC * NS
CH = 128
NCHUNK = N_EDGES // CH
NGFULL = 312
MAXG = 10
MAXCH = 80
RPS = 640
RLAST = N - 15 * RPS
NPD = 10240
NECH = 78

_mesh = plsc.VectorSubcoreMesh(
    core_axis_name="c", subcore_axis_name="s", num_cores=NC, num_subcores=NS)


def _wid():
  return lax.axis_index("s") * NC + lax.axis_index("c")


def _nchunks(w):
  return jnp.where(w < 24, 80, jnp.where(w < 28, 72, 73))


def _load_all_groups(edge3, which, buf, sem, w):
  for gi in range(MAXG):
    @pl.when(w + NW * gi < NGFULL)
    def _():
      pltpu.async_copy(
          edge3.at[which].at[pl.ds((w + NW * gi) * 8, 8)],
          buf.at[pl.ds(gi * 8, 8)], sem)
  @pl.when(w >= 28)
  def _():
    pltpu.async_copy(edge3.at[which].at[pl.ds(NGFULL * 8, 4)],
                     buf.at[pl.ds(72, 4)], sem)


def _drain_all_groups(edge3, buf, sem, w):
  ng = jnp.where(w < 24, MAXG, MAXG - 1)
  @pl.loop(0, ng)
  def _(i):
    pltpu.make_async_copy(edge3.at[0].at[pl.ds(0, 8)],
                          buf.at[pl.ds(0, 8)], sem).wait()
  @pl.when(w >= 28)
  def _():
    pltpu.make_async_copy(edge3.at[0].at[pl.ds(0, 4)],
                          buf.at[pl.ds(0, 4)], sem).wait()


def _load_pass_groups(edge3, which, buf, sem, w, p):
  for gl in range(5):
    gi = p * 5 + gl
    @pl.when(w + NW * gi < NGFULL)
    def _():
      pltpu.async_copy(
          edge3.at[which].at[pl.ds((w + NW * gi) * 8, 8)],
          buf.at[pl.ds(gl * 8, 8)], sem)
  if p == 1:
    @pl.when(w >= 28)
    def _():
      pltpu.async_copy(edge3.at[which].at[pl.ds(NGFULL * 8, 4)],
                       buf.at[pl.ds(32, 4)], sem)


def _drain_pass_groups(edge3, buf, sem, w, p, narrays):
  if p == 0:
    n = 5 * narrays
  else:
    n = narrays * jnp.where(w < 24, 5, 4)
  @pl.loop(0, n)
  def _(i):
    pltpu.make_async_copy(edge3.at[0].at[pl.ds(0, 8)],
                          buf.at[pl.ds(0, 8)], sem).wait()
  if p == 1:
    @pl.when(w >= 28)
    def _():
      for _k in range(narrays):
        pltpu.make_async_copy(edge3.at[0].at[pl.ds(0, 4)],
                              buf.at[pl.ds(0, 4)], sem).wait()


def _init_rows(dst_ref, src_full, src_last, s):
  @pl.when(s < 15)
  def _():
    pltpu.sync_copy(src_full, dst_ref.at[pl.ds(s * RPS, RPS)])
  @pl.when(s == 15)
  def _():
    pltpu.sync_copy(src_last, dst_ref.at[pl.ds(15 * RPS, RLAST)])


def _writeout_rows(src_ref, out_ref, s):
  @pl.when(s < 15)
  def _():
    pltpu.sync_copy(src_ref.at[pl.ds(s * RPS, RPS)],
                    out_ref.at[pl.ds(s * RPS, RPS)])
  @pl.when(s == 15)
  def _():
    pltpu.sync_copy(src_ref.at[pl.ds(15 * RPS, RLAST)],
                    out_ref.at[pl.ds(15 * RPS, RLAST)])


@functools.partial(
    pl.kernel,
    out_type=(
        jax.ShapeDtypeStruct((NPD,), jnp.float32),
        jax.ShapeDtypeStruct((NPD,), jnp.float32),
        jax.ShapeDtypeStruct((N, DIM), jnp.float32),
    ),
    mesh=_mesh,
    scratch_types=(
        pltpu.VMEM((MAXCH, CH), jnp.int32),
        pltpu.VMEM((CH,), jnp.float32),
        pltpu.VMEM((CH,), jnp.int32),
        pltpu.VMEM((CH, DIM), jnp.float32),
        pltpu.SemaphoreType.DMA,
        pltpu.SemaphoreType.DMA,
        pltpu.VMEM_SHARED((NPD,), jnp.float32),
    ),
)
def _sc_deg_embed(edge3, ones_hbm, zeros1_hbm, table_hbm, x_hbm,
                  deg0_out, deg1_out, h0_out,
                  dst_v, ones_v, x_v, rows_v, lsem, dsem, accd):
  c = lax.axis_index("c")
  s = lax.axis_index("s")
  w = _wid()

  _load_all_groups(edge3, 1, dst_v, lsem, w)
  @pl.when(c == 0)
  def _():
    pltpu.sync_copy(ones_hbm, accd.at[pl.ds(s * RPS, RPS)])
  @pl.when(c == 1)
  def _():
    pltpu.sync_copy(zeros1_hbm, accd.at[pl.ds(s * RPS, RPS)])
  pltpu.sync_copy(ones_hbm.at[pl.ds(0, CH)], ones_v)
  _drain_all_groups(edge3, dst_v, lsem, w)
  plsc.subcore_barrier()

  nch = _nchunks(w)

  @pl.loop(0, nch)
  def _count(j):
    row = j + jnp.where((j == 72) & (w >= 28), w - 28, 0)
    pltpu.async_copy(ones_v, accd.at[dst_v.at[row]], dsem, add=True)

  @pl.loop(w, NECH, step=NW)
  def _embed(t):
    pltpu.sync_copy(x_hbm.at[pl.ds(t * CH, CH)], x_v)
    pltpu.sync_copy(table_hbm.at[x_v], rows_v)
    pltpu.sync_copy(rows_v, h0_out.at[pl.ds(t * CH, CH)])

  @pl.when(w == 31)
  def _():
    pltpu.sync_copy(x_hbm.at[pl.ds(N - CH, CH)], x_v)
    pltpu.sync_copy(table_hbm.at[x_v.at[pl.ds(CH - 16, 16)]],
                    rows_v.at[pl.ds(0, 16)])
    pltpu.sync_copy(rows_v.at[pl.ds(0, 16)],
                    h0_out.at[pl.ds(NECH * CH, 16)])

  @pl.loop(0, nch)
  def _drain(j):
    pltpu.make_async_copy(ones_v, accd.at[pl.ds(0, CH)], dsem).wait()

  plsc.subcore_barrier()

  @pl.when(c == 0)
  def _():
    pltpu.sync_copy(accd.at[pl.ds(s * RPS, RPS)],
                    deg0_out.at[pl.ds(s * RPS, RPS)])

  @pl.when(c == 1)
  def _():
    pltpu.sync_copy(accd.at[pl.ds(s * RPS, RPS)],
                    deg1_out.at[pl.ds(s * RPS, RPS)])


NBUF = 2


@functools.partial(
    pl.kernel,
    out_type=jax.ShapeDtypeStruct((NC, N, DIM), jnp.float32),
    mesh=_mesh,
    scratch_types=(
        pltpu.VMEM((40, CH), jnp.int32),
        pltpu.VMEM((40, CH), jnp.int32),
        tuple(pltpu.VMEM((CH, DIM), jnp.float32) for _ in range(NBUF)),
        pltpu.SemaphoreType.DMA,
        tuple(pltpu.SemaphoreType.DMA for _ in range(NBUF)),
        tuple(pltpu.SemaphoreType.DMA for _ in range(NBUF)),
        pltpu.VMEM_SHARED((N, DIM), jnp.float32),
    ),
)
def _sc_edge_agg(edge3, zeros_hbm, g_hbm, acc_out, src_v, dst_v, rows,
                 lsem, gsem, ssem, acc):
  c = lax.axis_index("c")
  s = lax.axis_index("s")
  w = _wid()

  _load_pass_groups(edge3, 0, src_v, lsem, w, 0)
  _load_pass_groups(edge3, 1, dst_v, lsem, w, 0)
  @pl.when(c == 0)
  def _():
    _init_rows(acc, g_hbm.at[pl.ds(s * RPS, RPS)],
               g_hbm.at[pl.ds(15 * RPS, RLAST)], s)
  @pl.when(c == 1)
  def _():
    _init_rows(acc, zeros_hbm, zeros_hbm.at[pl.ds(0, RLAST)], s)
  _drain_pass_groups(edge3, dst_v, lsem, w, 0, 2)
  plsc.subcore_barrier()

  def gather(j, b):
    pltpu.async_copy(g_hbm.at[src_v.at[j]], rows[b], gsem[b])

  def gather_wait(b):
    pltpu.make_async_copy(g_hbm.at[pl.ds(0, CH)], rows[b], gsem[b]).wait()

  def scatter(j, b):
    pltpu.async_copy(rows[b], acc.at[dst_v.at[j]], ssem[b], add=True)

  def scatter_wait(b):
    pltpu.make_async_copy(rows[b], acc.at[pl.ds(0, CH)], ssem[b]).wait()

  def run_pipeline(npairs):
    gather(0, 0)

    @pl.loop(0, npairs - 1)
    def _agg(i):
      gather_wait(0)
      scatter(2 * i, 0)
      @pl.when(i > 0)
      def _():
        scatter_wait(1)
      gather(2 * i + 1, 1)
      gather_wait(1)
      scatter(2 * i + 1, 1)
      scatter_wait(0)
      gather(2 * i + 2, 0)

    last = 2 * (npairs - 1)
    gather_wait(0)
    scatter(last, 0)
    scatter_wait(1)
    gather(last + 1, 1)
    gather_wait(1)
    scatter(last + 1, 1)
    scatter_wait(0)
    scatter_wait(1)

  run_pipeline(20)

  _load_pass_groups(edge3, 0, src_v, lsem, w, 1)
  _load_pass_groups(edge3, 1, dst_v, lsem, w, 1)
  _drain_pass_groups(edge3, dst_v, lsem, w, 1, 2)
  run_pipeline(jnp.where(w < 24, 20, 16))

  @pl.when(w >= 28)
  def _():
    row = 32 + (w - 28)
    gather(row, 0)
    gather_wait(0)
    scatter(row, 0)
    scatter_wait(0)

  plsc.subcore_barrier()
  _writeout_rows(acc, acc_out.at[c], s)


def _dinv(deg0_ref, deg1_ref):
  return lax.rsqrt(deg0_ref[...][:N] + deg1_ref[...][:N])


def _tc_g1_body(h0_ref, x_ref, deg0_ref, deg1_ref, w1_ref, g1_ref):
  d = _dinv(deg0_ref, deg1_ref) * (x_ref[...] != 0).astype(jnp.float32)
  g1_ref[...] = d * jnp.dot(h0_ref[...], w1_ref[...],
                            preferred_element_type=jnp.float32)


def _tc_g2_body(p_ref, deg0_ref, deg1_ref, b1_ref, w2_ref, g2_ref):
  d = _dinv(deg0_ref, deg1_ref)
  p = p_ref[...]
  h1 = jnp.maximum(d * (p[0] + p[1]) + b1_ref[...], 0.0)
  g2_ref[...] = d * jnp.dot(h1, w2_ref[...],
                            preferred_element_type=jnp.float32)


def _tc_final_body(q_ref, deg0_ref, deg1_ref, b2_ref, batch_ref,
                   wlin_ref, blin_ref, out_ref):
  d = _dinv(deg0_ref, deg1_ref)
  q = q_ref[...]
  h2 = jnp.maximum(d * (q[0] + q[1]) + b2_ref[...], 0.0)
  gid = lax.broadcasted_iota(jnp.int32, (1, NUM_GRAPHS), 1)
  onehot = (batch_ref[...] == gid).astype(jnp.float32)
  cnt = jnp.sum(onehot, axis=0, keepdims=True)
  pooled = lax.dot_general(onehot, h2, (((0,), (0,)), ((), ())),
                           preferred_element_type=jnp.float32)
  pooled = pooled / jnp.maximum(cnt, 1.0).T
  out_ref[...] = jnp.dot(pooled, wlin_ref[...],
                         preferred_element_type=jnp.float32) + blin_ref[...]


_tc_g1 = pl.pallas_call(
    _tc_g1_body,
    out_shape=jax.ShapeDtypeStruct((N, DIM), jnp.float32))

_tc_g2 = pl.pallas_call(
    _tc_g2_body,
    out_shape=jax.ShapeDtypeStruct((N, DIM), jnp.float32))

_tc_final = pl.pallas_call(
    _tc_final_body,
    out_shape=jax.ShapeDtypeStruct((NUM_GRAPHS, NUM_CLASSES), jnp.float32))


@jax.jit
def kernel(x, edge_index, batch, emb_table, W1, b1, W2, b2, Wlin, blin):
  x = x.astype(jnp.int32)
  edge3 = edge_index.reshape(2, NCHUNK, CH)
  ones_c = jnp.ones((RPS,), jnp.float32)
  zeros1 = jnp.zeros((RPS,), jnp.float32)
  zrows = jnp.zeros((RPS, DIM), jnp.float32)

  deg0, deg1, h0 = _sc_deg_embed(edge3, ones_c, zeros1, emb_table, x)
  deg0 = deg0.reshape(NPD, 1)
  deg1 = deg1.reshape(NPD, 1)
  g1 = _tc_g1(h0, x.reshape(N, 1), deg0, deg1, W1)
  p1 = _sc_edge_agg(edge3, zrows, g1)
  g2 = _tc_g2(p1, deg0, deg1, b1.reshape(1, DIM), W2)
  p2 = _sc_edge_agg(edge3, zrows, g2)
  return _tc_final(p2, deg0, deg1, b2.reshape(1, DIM), batch.reshape(N, 1),
                   Wlin, blin.reshape(1, NUM_CLASSES))


# speedup vs baseline: 27.4323x; 1.0217x over previous
import functools

import jax
import jax.numpy as jnp
from jax import lax
from jax.experimental import pallas as pl
from jax.experimental.pallas import tpu as pltpu
from jax.experimental.pallas import tpu_sc as plsc

N = 10000
N_EDGES = 320000
VOCAB = 1000
DIM = 128
NUM_CLASSES = 10
NUM_GRAPHS = 64

NC, NS = 2, 16
NW = NC * NS
CH = 128
NCHUNK = N_EDGES // CH
NGFULL = 312
MAXG = 10
MAXCH = 80
RPS = 640
RLAST = N - 15 * RPS
NPD = 10240
NECH = 78

_mesh = plsc.VectorSubcoreMesh(
    core_axis_name="c", subcore_axis_name="s", num_cores=NC, num_subcores=NS)


def _wid():
  return lax.axis_index("s") * NC + lax.axis_index("c")


def _nchunks(w):
  return jnp.where(w < 24, 80, jnp.where(w < 28, 72, 73))


def _load_all_groups(edge3, which, buf, sem, w):
  for gi in range(MAXG):
    @pl.when(w + NW * gi < NGFULL)
    def _():
      pltpu.async_copy(
          edge3.at[which].at[pl.ds((w + NW * gi) * 8, 8)],
          buf.at[pl.ds(gi * 8, 8)], sem)
  @pl.when(w >= 28)
  def _():
    pltpu.async_copy(edge3.at[which].at[pl.ds(NGFULL * 8, 4)],
                     buf.at[pl.ds(72, 4)], sem)


def _drain_all_groups(edge3, buf, sem, w):
  ng = jnp.where(w < 24, MAXG, MAXG - 1)
  @pl.loop(0, ng)
  def _(i):
    pltpu.make_async_copy(edge3.at[0].at[pl.ds(0, 8)],
                          buf.at[pl.ds(0, 8)], sem).wait()
  @pl.when(w >= 28)
  def _():
    pltpu.make_async_copy(edge3.at[0].at[pl.ds(0, 4)],
                          buf.at[pl.ds(0, 4)], sem).wait()


def _load_pass_groups(edge3, which, buf, sem, w, p):
  for gl in range(5):
    gi = p * 5 + gl
    @pl.when(w + NW * gi < NGFULL)
    def _():
      pltpu.async_copy(
          edge3.at[which].at[pl.ds((w + NW * gi) * 8, 8)],
          buf.at[pl.ds(gl * 8, 8)], sem)
  if p == 1:
    @pl.when(w >= 28)
    def _():
      pltpu.async_copy(edge3.at[which].at[pl.ds(NGFULL * 8, 4)],
                       buf.at[pl.ds(32, 4)], sem)


def _drain_pass_groups(edge3, buf, sem, w, p, narrays):
  if p == 0:
    n = 5 * narrays
  else:
    n = narrays * jnp.where(w < 24, 5, 4)
  @pl.loop(0, n)
  def _(i):
    pltpu.make_async_copy(edge3.at[0].at[pl.ds(0, 8)],
                          buf.at[pl.ds(0, 8)], sem).wait()
  if p == 1:
    @pl.when(w >= 28)
    def _():
      for _k in range(narrays):
        pltpu.make_async_copy(edge3.at[0].at[pl.ds(0, 4)],
                              buf.at[pl.ds(0, 4)], sem).wait()


def _init_rows(dst_ref, src_full, src_last, s):
  @pl.when(s < 15)
  def _():
    pltpu.sync_copy(src_full, dst_ref.at[pl.ds(s * RPS, RPS)])
  @pl.when(s == 15)
  def _():
    pltpu.sync_copy(src_last, dst_ref.at[pl.ds(15 * RPS, RLAST)])


def _writeout_rows(src_ref, out_ref, s):
  @pl.when(s < 15)
  def _():
    pltpu.sync_copy(src_ref.at[pl.ds(s * RPS, RPS)],
                    out_ref.at[pl.ds(s * RPS, RPS)])
  @pl.when(s == 15)
  def _():
    pltpu.sync_copy(src_ref.at[pl.ds(15 * RPS, RLAST)],
                    out_ref.at[pl.ds(15 * RPS, RLAST)])


@functools.partial(
    pl.kernel,
    out_type=(
        jax.ShapeDtypeStruct((NPD,), jnp.float32),
        jax.ShapeDtypeStruct((NPD,), jnp.float32),
        jax.ShapeDtypeStruct((N, DIM), jnp.float32),
    ),
    mesh=_mesh,
    scratch_types=(
        pltpu.VMEM((MAXCH, CH), jnp.int32),
        pltpu.VMEM((CH,), jnp.float32),
        pltpu.VMEM((CH,), jnp.int32),
        pltpu.VMEM((CH, DIM), jnp.float32),
        pltpu.SemaphoreType.DMA,
        pltpu.SemaphoreType.DMA,
        pltpu.VMEM_SHARED((NPD,), jnp.float32),
    ),
)
def _sc_deg_embed(edge3, ones_hbm, zeros1_hbm, table_hbm, x_hbm,
                  deg0_out, deg1_out, h0_out,
                  dst_v, ones_v, x_v, rows_v, lsem, dsem, accd):
  c = lax.axis_index("c")
  s = lax.axis_index("s")
  w = _wid()

  _load_all_groups(edge3, 1, dst_v, lsem, w)
  @pl.when(c == 0)
  def _():
    pltpu.sync_copy(ones_hbm, accd.at[pl.ds(s * RPS, RPS)])
  @pl.when(c == 1)
  def _():
    pltpu.sync_copy(zeros1_hbm, accd.at[pl.ds(s * RPS, RPS)])
  pltpu.sync_copy(ones_hbm.at[pl.ds(0, CH)], ones_v)
  _drain_all_groups(edge3, dst_v, lsem, w)
  plsc.subcore_barrier()

  nch = _nchunks(w)

  @pl.loop(0, nch)
  def _count(j):
    row = j + jnp.where((j == 72) & (w >= 28), w - 28, 0)
    pltpu.async_copy(ones_v, accd.at[dst_v.at[row]], dsem, add=True)

  @pl.loop(w, NECH, step=NW)
  def _embed(t):
    pltpu.sync_copy(x_hbm.at[pl.ds(t * CH, CH)], x_v)
    pltpu.sync_copy(table_hbm.at[x_v], rows_v)
    pltpu.sync_copy(rows_v, h0_out.at[pl.ds(t * CH, CH)])

  @pl.when(w == 31)
  def _():
    pltpu.sync_copy(x_hbm.at[pl.ds(N - CH, CH)], x_v)
    pltpu.sync_copy(table_hbm.at[x_v.at[pl.ds(CH - 16, 16)]],
                    rows_v.at[pl.ds(0, 16)])
    pltpu.sync_copy(rows_v.at[pl.ds(0, 16)],
                    h0_out.at[pl.ds(NECH * CH, 16)])

  @pl.loop(0, nch)
  def _drain(j):
    pltpu.make_async_copy(ones_v, accd.at[pl.ds(0, CH)], dsem).wait()

  plsc.subcore_barrier()

  @pl.when(c == 0)
  def _():
    pltpu.sync_copy(accd.at[pl.ds(s * RPS, RPS)],
                    deg0_out.at[pl.ds(s * RPS, RPS)])

  @pl.when(c == 1)
  def _():
    pltpu.sync_copy(accd.at[pl.ds(s * RPS, RPS)],
                    deg1_out.at[pl.ds(s * RPS, RPS)])


CHA = 64
NBUF = 4
AG = 625


@functools.partial(
    pl.kernel,
    out_type=jax.ShapeDtypeStruct((NC, N, DIM), jnp.float32),
    mesh=_mesh,
    scratch_types=(
        pltpu.VMEM((40, CHA), jnp.int32),
        pltpu.VMEM((40, CHA), jnp.int32),
        tuple(pltpu.VMEM((CHA, DIM), jnp.float32) for _ in range(NBUF)),
        pltpu.SemaphoreType.DMA,
        tuple(pltpu.SemaphoreType.DMA for _ in range(NBUF)),
        tuple(pltpu.SemaphoreType.DMA for _ in range(NBUF)),
        pltpu.VMEM_SHARED((N, DIM), jnp.float32),
    ),
)
def _sc_edge_agg(edge3, zeros_hbm, g_hbm, acc_out, src_v, dst_v, rows,
                 lsem, gsem, ssem, acc):
  c = lax.axis_index("c")
  s = lax.axis_index("s")
  w = _wid()

  def load_pass(p):
    for gl in range(5):
      gi = 5 * p + gl
      @pl.when(w + NW * gi < AG)
      def _():
        for which, buf in ((0, src_v), (1, dst_v)):
          pltpu.async_copy(
              edge3.at[which].at[pl.ds((w + NW * gi) * 8, 8)],
              buf.at[pl.ds(gl * 8, 8)], lsem)

  def drain_pass(p):
    n = 10 if p < 3 else 2 * jnp.where(w < 17, 5, 4)
    @pl.loop(0, n)
    def _(i):
      pltpu.make_async_copy(edge3.at[0].at[pl.ds(0, 8)],
                            src_v.at[pl.ds(0, 8)], lsem).wait()

  load_pass(0)
  @pl.when(c == 0)
  def _():
    _init_rows(acc, g_hbm.at[pl.ds(s * RPS, RPS)],
               g_hbm.at[pl.ds(15 * RPS, RLAST)], s)
  @pl.when(c == 1)
  def _():
    _init_rows(acc, zeros_hbm, zeros_hbm.at[pl.ds(0, RLAST)], s)
  drain_pass(0)
  plsc.subcore_barrier()

  def gather(j, b):
    pltpu.async_copy(g_hbm.at[src_v.at[j]], rows[b], gsem[b])

  def gather_wait(b):
    pltpu.make_async_copy(g_hbm.at[pl.ds(0, CHA)], rows[b], gsem[b]).wait()

  def scatter(j, b):
    pltpu.async_copy(rows[b], acc.at[dst_v.at[j]], ssem[b], add=True)

  def scatter_wait(b):
    pltpu.make_async_copy(rows[b], acc.at[pl.ds(0, CHA)], ssem[b]).wait()

  def run_pipeline(nq):
    gather(0, 0)
    gather(1, 1)
    for b in range(4):
      gather_wait(b)
      scatter(b, b)
      if b >= 2:
        scatter_wait(b - 2)
      gather(b + 2, (b + 2) % 4)

    @pl.loop(1, nq - 1)
    def _agg(i):
      base = 4 * i
      for b in range(4):
        gather_wait(b)
        scatter(base + b, b)
        scatter_wait((b + 2) % 4)
        gather(base + b + 2, (b + 2) % 4)

    last = 4 * (nq - 1)
    for b in range(4):
      gather_wait(b)
      scatter(last + b, b)
      scatter_wait((b + 2) % 4)
      if b < 2:
        gather(last + b + 2, (b + 2) % 4)
    for b in range(2, 4):
      scatter_wait(b)

  run_pipeline(10)
  for p in (1, 2):
    load_pass(p)
    drain_pass(p)
    run_pipeline(10)
  load_pass(3)
  drain_pass(3)
  run_pipeline(jnp.where(w < 17, 10, 8))

  plsc.subcore_barrier()
  _writeout_rows(acc, acc_out.at[c], s)


def _dinv(deg0_ref, deg1_ref):
  return lax.rsqrt(deg0_ref[...][:N] + deg1_ref[...][:N])


def _tc_g1_body(h0_ref, x_ref, deg0_ref, deg1_ref, w1_ref, g1_ref):
  d = _dinv(deg0_ref, deg1_ref) * (x_ref[...] != 0).astype(jnp.float32)
  g1_ref[...] = d * jnp.dot(h0_ref[...], w1_ref[...],
                            preferred_element_type=jnp.float32)


def _tc_g2_body(p_ref, deg0_ref, deg1_ref, b1_ref, w2_ref, g2_ref):
  d = _dinv(deg0_ref, deg1_ref)
  p = p_ref[...]
  h1 = jnp.maximum(d * (p[0] + p[1]) + b1_ref[...], 0.0)
  g2_ref[...] = d * jnp.dot(h1, w2_ref[...],
                            preferred_element_type=jnp.float32)


def _tc_final_body(q_ref, deg0_ref, deg1_ref, b2_ref, batch_ref,
                   wlin_ref, blin_ref, out_ref):
  d = _dinv(deg0_ref, deg1_ref)
  q = q_ref[...]
  h2 = jnp.maximum(d * (q[0] + q[1]) + b2_ref[...], 0.0)
  gid = lax.broadcasted_iota(jnp.int32, (1, NUM_GRAPHS), 1)
  onehot = (batch_ref[...] == gid).astype(jnp.float32)
  cnt = jnp.sum(onehot, axis=0, keepdims=True)
  pooled = lax.dot_general(onehot, h2, (((0,), (0,)), ((), ())),
                           preferred_element_type=jnp.float32)
  pooled = pooled / jnp.maximum(cnt, 1.0).T
  out_ref[...] = jnp.dot(pooled, wlin_ref[...],
                         preferred_element_type=jnp.float32) + blin_ref[...]


_tc_g1 = pl.pallas_call(
    _tc_g1_body,
    out_shape=jax.ShapeDtypeStruct((N, DIM), jnp.float32))

_tc_g2 = pl.pallas_call(
    _tc_g2_body,
    out_shape=jax.ShapeDtypeStruct((N, DIM), jnp.float32))

_tc_final = pl.pallas_call(
    _tc_final_body,
    out_shape=jax.ShapeDtypeStruct((NUM_GRAPHS, NUM_CLASSES), jnp.float32))


@jax.jit
def kernel(x, edge_index, batch, emb_table, W1, b1, W2, b2, Wlin, blin):
  x = x.astype(jnp.int32)
  edge3 = edge_index.reshape(2, NCHUNK, CH)
  ones_c = jnp.ones((RPS,), jnp.float32)
  zeros1 = jnp.zeros((RPS,), jnp.float32)
  zrows = jnp.zeros((RPS, DIM), jnp.float32)

  edge3b = edge_index.reshape(2, 5000, 64)
  deg0, deg1, h0 = _sc_deg_embed(edge3, ones_c, zeros1, emb_table, x)
  deg0 = deg0.reshape(NPD, 1)
  deg1 = deg1.reshape(NPD, 1)
  g1 = _tc_g1(h0, x.reshape(N, 1), deg0, deg1, W1)
  p1 = _sc_edge_agg(edge3b, zrows, g1)
  g2 = _tc_g2(p1, deg0, deg1, b1.reshape(1, DIM), W2)
  p2 = _sc_edge_agg(edge3b, zrows, g2)
  return _tc_final(p2, deg0, deg1, b2.reshape(1, DIM), batch.reshape(N, 1),
                   Wlin, blin.reshape(1, NUM_CLASSES))


# speedup vs baseline: 30.3060x; 1.1048x over previous
import functools

import jax
import jax.numpy as jnp
from jax import lax
from jax.experimental import pallas as pl
from jax.experimental.pallas import tpu as pltpu
from jax.experimental.pallas import tpu_sc as plsc

N = 10000
N_EDGES = 320000
VOCAB = 1000
DIM = 128
NUM_CLASSES = 10
NUM_GRAPHS = 64

NC, NS = 2, 16
NW = NC * NS
CH = 128
NCHUNK = N_EDGES // CH
NGFULL = 312
MAXG = 10
MAXCH = 80
RPS = 640
RLAST = N - 15 * RPS
NPD = 10240
NECH = 78

_mesh = plsc.VectorSubcoreMesh(
    core_axis_name="c", subcore_axis_name="s", num_cores=NC, num_subcores=NS)


def _wid():
  return lax.axis_index("s") * NC + lax.axis_index("c")


def _nchunks(w):
  return jnp.where(w < 24, 80, jnp.where(w < 28, 72, 73))


def _load_all_groups(edge3, which, buf, sem, w):
  for gi in range(MAXG):
    @pl.when(w + NW * gi < NGFULL)
    def _():
      pltpu.async_copy(
          edge3.at[which].at[pl.ds((w + NW * gi) * 8, 8)],
          buf.at[pl.ds(gi * 8, 8)], sem)
  @pl.when(w >= 28)
  def _():
    pltpu.async_copy(edge3.at[which].at[pl.ds(NGFULL * 8, 4)],
                     buf.at[pl.ds(72, 4)], sem)


def _drain_all_groups(edge3, buf, sem, w):
  ng = jnp.where(w < 24, MAXG, MAXG - 1)
  @pl.loop(0, ng)
  def _(i):
    pltpu.make_async_copy(edge3.at[0].at[pl.ds(0, 8)],
                          buf.at[pl.ds(0, 8)], sem).wait()
  @pl.when(w >= 28)
  def _():
    pltpu.make_async_copy(edge3.at[0].at[pl.ds(0, 4)],
                          buf.at[pl.ds(0, 4)], sem).wait()


def _load_pass_groups(edge3, which, buf, sem, w, p):
  for gl in range(5):
    gi = p * 5 + gl
    @pl.when(w + NW * gi < NGFULL)
    def _():
      pltpu.async_copy(
          edge3.at[which].at[pl.ds((w + NW * gi) * 8, 8)],
          buf.at[pl.ds(gl * 8, 8)], sem)
  if p == 1:
    @pl.when(w >= 28)
    def _():
      pltpu.async_copy(edge3.at[which].at[pl.ds(NGFULL * 8, 4)],
                       buf.at[pl.ds(32, 4)], sem)


def _drain_pass_groups(edge3, buf, sem, w, p, narrays):
  if p == 0:
    n = 5 * narrays
  else:
    n = narrays * jnp.where(w < 24, 5, 4)
  @pl.loop(0, n)
  def _(i):
    pltpu.make_async_copy(edge3.at[0].at[pl.ds(0, 8)],
                          buf.at[pl.ds(0, 8)], sem).wait()
  if p == 1:
    @pl.when(w >= 28)
    def _():
      for _k in range(narrays):
        pltpu.make_async_copy(edge3.at[0].at[pl.ds(0, 4)],
                              buf.at[pl.ds(0, 4)], sem).wait()


def _init_rows(dst_ref, src_full, src_last, s):
  @pl.when(s < 15)
  def _():
    pltpu.sync_copy(src_full, dst_ref.at[pl.ds(s * RPS, RPS)])
  @pl.when(s == 15)
  def _():
    pltpu.sync_copy(src_last, dst_ref.at[pl.ds(15 * RPS, RLAST)])


def _writeout_rows(src_ref, out_ref, s):
  @pl.when(s < 15)
  def _():
    pltpu.sync_copy(src_ref.at[pl.ds(s * RPS, RPS)],
                    out_ref.at[pl.ds(s * RPS, RPS)])
  @pl.when(s == 15)
  def _():
    pltpu.sync_copy(src_ref.at[pl.ds(15 * RPS, RLAST)],
                    out_ref.at[pl.ds(15 * RPS, RLAST)])


@functools.partial(
    pl.kernel,
    out_type=(
        jax.ShapeDtypeStruct((NPD,), jnp.float32),
        jax.ShapeDtypeStruct((NPD,), jnp.float32),
        jax.ShapeDtypeStruct((N, DIM), jnp.float32),
    ),
    mesh=_mesh,
    scratch_types=(
        pltpu.VMEM((MAXCH, CH), jnp.int32),
        pltpu.VMEM((CH,), jnp.float32),
        pltpu.VMEM((CH,), jnp.int32),
        pltpu.VMEM((CH, DIM), jnp.float32),
        pltpu.SemaphoreType.DMA,
        pltpu.SemaphoreType.DMA,
        pltpu.VMEM_SHARED((NPD,), jnp.float32),
    ),
)
def _sc_deg_embed(edge3, ones_hbm, zeros1_hbm, table_hbm, x_hbm,
                  deg0_out, deg1_out, h0_out,
                  dst_v, ones_v, x_v, rows_v, lsem, dsem, accd):
  c = lax.axis_index("c")
  s = lax.axis_index("s")
  w = _wid()

  _load_all_groups(edge3, 1, dst_v, lsem, w)
  @pl.when(c == 0)
  def _():
    pltpu.sync_copy(ones_hbm, accd.at[pl.ds(s * RPS, RPS)])
  @pl.when(c == 1)
  def _():
    pltpu.sync_copy(zeros1_hbm, accd.at[pl.ds(s * RPS, RPS)])
  pltpu.sync_copy(ones_hbm.at[pl.ds(0, CH)], ones_v)
  _drain_all_groups(edge3, dst_v, lsem, w)
  plsc.subcore_barrier()

  nch = _nchunks(w)

  @pl.loop(0, nch)
  def _count(j):
    row = j + jnp.where((j == 72) & (w >= 28), w - 28, 0)
    pltpu.async_copy(ones_v, accd.at[dst_v.at[row]], dsem, add=True)

  @pl.loop(w, NECH, step=NW)
  def _embed(t):
    pltpu.sync_copy(x_hbm.at[pl.ds(t * CH, CH)], x_v)
    pltpu.sync_copy(table_hbm.at[x_v], rows_v)
    pltpu.sync_copy(rows_v, h0_out.at[pl.ds(t * CH, CH)])

  @pl.when(w == 31)
  def _():
    pltpu.sync_copy(x_hbm.at[pl.ds(N - CH, CH)], x_v)
    pltpu.sync_copy(table_hbm.at[x_v.at[pl.ds(CH - 16, 16)]],
                    rows_v.at[pl.ds(0, 16)])
    pltpu.sync_copy(rows_v.at[pl.ds(0, 16)],
                    h0_out.at[pl.ds(NECH * CH, 16)])

  @pl.loop(0, nch)
  def _drain(j):
    pltpu.make_async_copy(ones_v, accd.at[pl.ds(0, CH)], dsem).wait()

  plsc.subcore_barrier()

  @pl.when(c == 0)
  def _():
    pltpu.sync_copy(accd.at[pl.ds(s * RPS, RPS)],
                    deg0_out.at[pl.ds(s * RPS, RPS)])

  @pl.when(c == 1)
  def _():
    pltpu.sync_copy(accd.at[pl.ds(s * RPS, RPS)],
                    deg1_out.at[pl.ds(s * RPS, RPS)])


CHA = 64
NBUF = 4
AG = 625


@functools.partial(
    pl.kernel,
    out_type=jax.ShapeDtypeStruct((NC, N, DIM), jnp.float32),
    mesh=_mesh,
    scratch_types=(
        pltpu.VMEM((40, CHA), jnp.int32),
        pltpu.VMEM((40, CHA), jnp.int32),
        tuple(pltpu.VMEM((CHA, DIM), jnp.float32) for _ in range(NBUF)),
        pltpu.SemaphoreType.DMA,
        tuple(pltpu.SemaphoreType.DMA for _ in range(NBUF)),
        tuple(pltpu.SemaphoreType.DMA for _ in range(NBUF)),
        pltpu.VMEM_SHARED((N, DIM), jnp.float32),
    ),
)
def _sc_edge_agg(edge3, zeros_hbm, g_hbm, acc_out, src_v, dst_v, rows,
                 lsem, gsem, ssem, acc):
  c = lax.axis_index("c")
  s = lax.axis_index("s")
  w = _wid()

  def load_pass(p):
    for gl in range(5):
      gi = 5 * p + gl
      @pl.when(w + NW * gi < AG)
      def _():
        for which, buf in ((0, src_v), (1, dst_v)):
          pltpu.async_copy(
              edge3.at[which].at[pl.ds((w + NW * gi) * 8, 8)],
              buf.at[pl.ds(gl * 8, 8)], lsem)

  def drain_pass(p):
    n = 10 if p < 3 else 2 * jnp.where(w < 17, 5, 4)
    @pl.loop(0, n)
    def _(i):
      pltpu.make_async_copy(edge3.at[0].at[pl.ds(0, 8)],
                            src_v.at[pl.ds(0, 8)], lsem).wait()

  load_pass(0)
  @pl.when(c == 0)
  def _():
    _init_rows(acc, g_hbm.at[pl.ds(s * RPS, RPS)],
               g_hbm.at[pl.ds(15 * RPS, RLAST)], s)
  @pl.when(c == 1)
  def _():
    _init_rows(acc, zeros_hbm, zeros_hbm.at[pl.ds(0, RLAST)], s)
  drain_pass(0)
  plsc.subcore_barrier()

  def gather(j, b):
    pltpu.async_copy(g_hbm.at[src_v.at[j]], rows[b], gsem[b])

  def gather_wait(b):
    pltpu.make_async_copy(g_hbm.at[pl.ds(0, CHA)], rows[b], gsem[b]).wait()

  def scatter(j, b):
    pltpu.async_copy(rows[b], acc.at[dst_v.at[j]], ssem[b], add=True)

  def scatter_wait(b):
    pltpu.make_async_copy(rows[b], acc.at[pl.ds(0, CHA)], ssem[b]).wait()

  def run_pipeline(nq):
    for b in range(3):
      gather(b, b)
    for b in range(4):
      gather_wait(b)
      scatter(b, b)
      if b >= 1:
        scatter_wait(b - 1)
      gather(b + 3, (b + 3) % 4)

    @pl.loop(1, nq - 1)
    def _agg(i):
      base = 4 * i
      for b in range(4):
        gather_wait(b)
        scatter(base + b, b)
        scatter_wait((b + 3) % 4)
        gather(base + b + 3, (b + 3) % 4)

    last = 4 * (nq - 1)
    for b in range(4):
      gather_wait(b)
      scatter(last + b, b)
      scatter_wait((b + 3) % 4)
      if b == 0:
        gather(last + 3, 3)
    scatter_wait(3)

  run_pipeline(10)
  for p in (1, 2):
    load_pass(p)
    drain_pass(p)
    run_pipeline(10)
  load_pass(3)
  drain_pass(3)
  run_pipeline(jnp.where(w < 17, 10, 8))

  plsc.subcore_barrier()
  _writeout_rows(acc, acc_out.at[c], s)


def _dinv(deg0_ref, deg1_ref):
  return lax.rsqrt(deg0_ref[...][:N] + deg1_ref[...][:N])


def _tc_g1_body(h0_ref, x_ref, deg0_ref, deg1_ref, w1_ref, g1_ref):
  d = _dinv(deg0_ref, deg1_ref) * (x_ref[...] != 0).astype(jnp.float32)
  g1_ref[...] = d * jnp.dot(h0_ref[...], w1_ref[...],
                            preferred_element_type=jnp.float32)


def _tc_g2_body(p_ref, deg0_ref, deg1_ref, b1_ref, w2_ref, g2_ref):
  d = _dinv(deg0_ref, deg1_ref)
  p = p_ref[...]
  h1 = jnp.maximum(d * (p[0] + p[1]) + b1_ref[...], 0.0)
  g2_ref[...] = d * jnp.dot(h1, w2_ref[...],
                            preferred_element_type=jnp.float32)


def _tc_final_body(q_ref, deg0_ref, deg1_ref, b2_ref, batch_ref,
                   wlin_ref, blin_ref, out_ref):
  d = _dinv(deg0_ref, deg1_ref)
  q = q_ref[...]
  h2 = jnp.maximum(d * (q[0] + q[1]) + b2_ref[...], 0.0)
  gid = lax.broadcasted_iota(jnp.int32, (1, NUM_GRAPHS), 1)
  onehot = (batch_ref[...] == gid).astype(jnp.float32)
  cnt = jnp.sum(onehot, axis=0, keepdims=True)
  pooled = lax.dot_general(onehot, h2, (((0,), (0,)), ((), ())),
                           preferred_element_type=jnp.float32)
  pooled = pooled / jnp.maximum(cnt, 1.0).T
  out_ref[...] = jnp.dot(pooled, wlin_ref[...],
                         preferred_element_type=jnp.float32) + blin_ref[...]


_tc_g1 = pl.pallas_call(
    _tc_g1_body,
    out_shape=jax.ShapeDtypeStruct((N, DIM), jnp.float32))

_tc_g2 = pl.pallas_call(
    _tc_g2_body,
    out_shape=jax.ShapeDtypeStruct((N, DIM), jnp.float32))

_tc_final = pl.pallas_call(
    _tc_final_body,
    out_shape=jax.ShapeDtypeStruct((NUM_GRAPHS, NUM_CLASSES), jnp.float32))


@jax.jit
def kernel(x, edge_index, batch, emb_table, W1, b1, W2, b2, Wlin, blin):
  x = x.astype(jnp.int32)
  edge3 = edge_index.reshape(2, NCHUNK, CH)
  ones_c = jnp.ones((RPS,), jnp.float32)
  zeros1 = jnp.zeros((RPS,), jnp.float32)
  zrows = jnp.zeros((RPS, DIM), jnp.float32)

  edge3b = edge_index.reshape(2, 5000, 64)
  deg0, deg1, h0 = _sc_deg_embed(edge3, ones_c, zeros1, emb_table, x)
  deg0 = deg0.reshape(NPD, 1)
  deg1 = deg1.reshape(NPD, 1)
  g1 = _tc_g1(h0, x.reshape(N, 1), deg0, deg1, W1)
  p1 = _sc_edge_agg(edge3b, zrows, g1)
  g2 = _tc_g2(p1, deg0, deg1, b1.reshape(1, DIM), W2)
  p2 = _sc_edge_agg(edge3b, zrows, g2)
  return _tc_final(p2, deg0, deg1, b2.reshape(1, DIM), batch.reshape(N, 1),
                   Wlin, blin.reshape(1, NUM_CLASSES))
